# baseline (speedup 1.0000x reference)
import jax
import jax.numpy as jnp
from jax.experimental import pallas as pl

_N_G = 256
_N_C = 64
_K = 16
_k = 4
_n_G = 16
_B = 8
_N = 8192


def _mlp_pallas(x, W1, b1, W2, b2, row_block=None):
    R, Din = x.shape
    H = W1.shape[1]
    Dout = W2.shape[1]
    if row_block is None:
        row_block = R

    def body(x_ref, w1_ref, b1_ref, w2_ref, b2_ref, o_ref):
        h = jnp.dot(x_ref[...], w1_ref[...], preferred_element_type=jnp.float32)
        h = jnp.maximum(h + b1_ref[...], 0.0)
        o_ref[...] = (
            jnp.dot(h, w2_ref[...], preferred_element_type=jnp.float32) + b2_ref[...]
        )

    return pl.pallas_call(
        body,
        grid=(R // row_block,),
        in_specs=[
            pl.BlockSpec((row_block, Din), lambda i: (i, 0)),
            pl.BlockSpec((Din, H), lambda i: (0, 0)),
            pl.BlockSpec((1, H), lambda i: (0, 0)),
            pl.BlockSpec((H, Dout), lambda i: (0, 0)),
            pl.BlockSpec((1, Dout), lambda i: (0, 0)),
        ],
        out_specs=pl.BlockSpec((row_block, Dout), lambda i: (i, 0)),
        out_shape=jax.ShapeDtypeStruct((R, Dout), jnp.float32),
    )(x, W1, b1.reshape(1, -1), W2, b2.reshape(1, -1))


def _fps_idx(pts, n):
    def one(p):
        Np = p.shape[0]
        dists0 = jnp.full((Np,), 1e10, dtype=p.dtype)
        idxs0 = jnp.zeros((n,), dtype=jnp.int32).at[0].set(0)

        def body(i, carry):
            idxs, dists, last = carry
            d = jnp.sum((p - p[last]) ** 2, axis=-1)
            dists = jnp.minimum(dists, d)
            nxt = jnp.argmax(dists).astype(jnp.int32)
            idxs = idxs.at[i].set(nxt)
            return (idxs, dists, nxt)

        idxs, _, _ = jax.lax.fori_loop(1, n, body, (idxs0, dists0, jnp.int32(0)))
        return idxs

    return jax.vmap(one)(pts)


def _gather_pts(p, idx):
    def one(a, i):
        return a[i.reshape(-1)].reshape(i.shape + (3,))
    return jax.vmap(one)(p, idx)


def kernel(pts, Wg1, bg1, Wg2, bg2, Wl1, bl1, Wl2, bl2):
    Bb = pts.shape[0]
    fidx = _fps_idx(pts, _N_G)
    pts_g = _gather_pts(pts, fidx)

    rec_g = _mlp_pallas(pts_g.reshape(Bb * _N_G, 3), Wg1, bg1, Wg2, bg2)
    rec_g = rec_g.reshape(Bb, _N_G, 3)

    d = jnp.sum((rec_g[:, :, None, :] - pts[:, None, :, :]) ** 2, axis=-1)
    _, knn_idx = jax.lax.top_k(-d, _N_C)
    pts_c = _gather_pts(pts, knn_idx)

    center = jnp.mean(pts_c, axis=2, keepdims=True)
    centered = pts_c - center
    scale = jnp.max(
        jnp.linalg.norm(centered, axis=-1, keepdims=True), axis=2, keepdims=True
    ) + 1e-9
    pts_c_n = centered / scale

    e = _mlp_pallas(
        pts_c_n.reshape(Bb * _N_G * _N_C, 3), Wl1, bl1, Wl2, bl2, row_block=16384
    )
    e = e.reshape(Bb * _N_G, _N_C, 2)

    eps = 1e-6
    mn = jnp.min(e, axis=1, keepdims=True)
    mx = jnp.max(e, axis=1, keepdims=True)
    e = (e - mn) / (mx - mn + 1e-12) * (1.0 - 2.0 * eps) + eps
    ebd_c = e.reshape(Bb, _N_G, _N_C, 2)

    g = (jnp.arange(_k, dtype=jnp.float32) + 0.5) / _k
    gx, gy = jnp.meshgrid(g, g, indexing="ij")
    grid = jnp.stack([gx.reshape(-1), gy.reshape(-1)], axis=-1)
    gd = jnp.sum(
        (grid[None, None, :, None, :] - ebd_c[:, :, None, :, :]) ** 2, axis=-1
    )
    nn_idx = jnp.argmin(gd, axis=-1)
    idx3 = jnp.broadcast_to(nn_idx[..., None], nn_idx.shape + (3,))
    pgi_local = jnp.take_along_axis(pts_c, idx3, axis=2)

    pg = pgi_local.reshape(Bb, _n_G, _n_G, _k, _k, 3)
    pg = jnp.transpose(pg, (0, 1, 3, 2, 4, 5)).reshape(Bb, _n_G * _k, _n_G * _k, 3)
    pgi = jnp.transpose(pg, (0, 3, 1, 2)).reshape(Bb, 3, _N_G * _K)
    pgi = jnp.transpose(pgi, (0, 2, 1))
    return pgi


# speedup vs baseline: 1.6041x; 1.6041x over previous
import jax
import jax.numpy as jnp
from jax.experimental import pallas as pl

_N_G = 256
_N_C = 64
_K = 16
_k = 4
_n_G = 16
_B = 8
_N = 8192


def _mlp_pallas(x, W1, b1, W2, b2, row_block=None):
    R, Din = x.shape
    H = W1.shape[1]
    Dout = W2.shape[1]
    if row_block is None:
        row_block = R

    def body(x_ref, w1_ref, b1_ref, w2_ref, b2_ref, o_ref):
        h = jnp.dot(x_ref[...], w1_ref[...], preferred_element_type=jnp.float32)
        h = jnp.maximum(h + b1_ref[...], 0.0)
        o_ref[...] = (
            jnp.dot(h, w2_ref[...], preferred_element_type=jnp.float32) + b2_ref[...]
        )

    return pl.pallas_call(
        body,
        grid=(R // row_block,),
        in_specs=[
            pl.BlockSpec((row_block, Din), lambda i: (i, 0)),
            pl.BlockSpec((Din, H), lambda i: (0, 0)),
            pl.BlockSpec((1, H), lambda i: (0, 0)),
            pl.BlockSpec((H, Dout), lambda i: (0, 0)),
            pl.BlockSpec((1, Dout), lambda i: (0, 0)),
        ],
        out_specs=pl.BlockSpec((row_block, Dout), lambda i: (i, 0)),
        out_shape=jax.ShapeDtypeStruct((R, Dout), jnp.float32),
    )(x, W1, b1.reshape(1, -1), W2, b2.reshape(1, -1))


def _fps_recg_pallas(pts, Wg1, bg1, Wg2, bg2):
    Bb = pts.shape[0]
    R = _N // 128
    ptsr = jnp.transpose(pts, (0, 2, 1)).reshape(Bb, 3, R, 128)

    def body(p_ref, w1_ref, b1_ref, w2_ref, b2_ref, g_ref, r_ref):
        px = p_ref[:, 0]
        py = p_ref[:, 1]
        pz = p_ref[:, 2]
        flat = (
            jax.lax.broadcasted_iota(jnp.int32, (Bb, R, 128), 1) * 128
            + jax.lax.broadcasted_iota(jnp.int32, (Bb, R, 128), 2)
        )

        def step(i, carry):
            dists, lx, ly, lz = carry
            d = (px - lx) ** 2 + (py - ly) ** 2 + (pz - lz) ** 2
            dists = jnp.minimum(dists, d)
            m = jnp.max(dists, axis=(1, 2), keepdims=True)
            cand = jnp.where(dists == m, flat, _N)
            nxt = jnp.min(cand, axis=(1, 2), keepdims=True)
            sel = flat == nxt
            nlx = jnp.sum(jnp.where(sel, px, 0.0), axis=(1, 2), keepdims=True)
            nly = jnp.sum(jnp.where(sel, py, 0.0), axis=(1, 2), keepdims=True)
            nlz = jnp.sum(jnp.where(sel, pz, 0.0), axis=(1, 2), keepdims=True)
            row = jnp.concatenate([nlx, nly, nlz], axis=2)
            g_ref[:, pl.ds(i, 1), :] = row
            return dists, nlx, nly, nlz

        lx0 = px[:, 0:1, 0:1]
        ly0 = py[:, 0:1, 0:1]
        lz0 = pz[:, 0:1, 0:1]
        g_ref[:, 0:1, :] = jnp.concatenate([lx0, ly0, lz0], axis=2)
        dists0 = jnp.full((Bb, R, 128), 1e10, jnp.float32)
        jax.lax.fori_loop(1, _N_G, step, (dists0, lx0, ly0, lz0), unroll=2)

        ptsg = g_ref[...].reshape(Bb * _N_G, 3)
        h = jnp.dot(ptsg, w1_ref[...], preferred_element_type=jnp.float32)
        h = jnp.maximum(h + b1_ref[...], 0.0)
        rec = jnp.dot(h, w2_ref[...], preferred_element_type=jnp.float32) + b2_ref[...]
        r_ref[...] = rec.reshape(Bb, _N_G, 3)

    return pl.pallas_call(
        body,
        out_shape=(
            jax.ShapeDtypeStruct((Bb, _N_G, 3), jnp.float32),
            jax.ShapeDtypeStruct((Bb, _N_G, 3), jnp.float32),
        ),
    )(ptsr, Wg1, bg1.reshape(1, -1), Wg2, bg2.reshape(1, -1))


def _gather_pts(p, idx):
    def one(a, i):
        return a[i.reshape(-1)].reshape(i.shape + (3,))
    return jax.vmap(one)(p, idx)


def kernel(pts, Wg1, bg1, Wg2, bg2, Wl1, bl1, Wl2, bl2):
    Bb = pts.shape[0]
    pts_g, rec_g = _fps_recg_pallas(pts, Wg1, bg1, Wg2, bg2)

    d = jnp.sum((rec_g[:, :, None, :] - pts[:, None, :, :]) ** 2, axis=-1)
    _, knn_idx = jax.lax.top_k(-d, _N_C)
    pts_c = _gather_pts(pts, knn_idx)

    center = jnp.mean(pts_c, axis=2, keepdims=True)
    centered = pts_c - center
    scale = jnp.max(
        jnp.linalg.norm(centered, axis=-1, keepdims=True), axis=2, keepdims=True
    ) + 1e-9
    pts_c_n = centered / scale

    e = _mlp_pallas(
        pts_c_n.reshape(Bb * _N_G * _N_C, 3), Wl1, bl1, Wl2, bl2, row_block=16384
    )
    e = e.reshape(Bb * _N_G, _N_C, 2)

    eps = 1e-6
    mn = jnp.min(e, axis=1, keepdims=True)
    mx = jnp.max(e, axis=1, keepdims=True)
    e = (e - mn) / (mx - mn + 1e-12) * (1.0 - 2.0 * eps) + eps
    ebd_c = e.reshape(Bb, _N_G, _N_C, 2)

    g = (jnp.arange(_k, dtype=jnp.float32) + 0.5) / _k
    gx, gy = jnp.meshgrid(g, g, indexing="ij")
    grid = jnp.stack([gx.reshape(-1), gy.reshape(-1)], axis=-1)
    gd = jnp.sum(
        (grid[None, None, :, None, :] - ebd_c[:, :, None, :, :]) ** 2, axis=-1
    )
    nn_idx = jnp.argmin(gd, axis=-1)
    idx3 = jnp.broadcast_to(nn_idx[..., None], nn_idx.shape + (3,))
    pgi_local = jnp.take_along_axis(pts_c, idx3, axis=2)

    pg = pgi_local.reshape(Bb, _n_G, _n_G, _k, _k, 3)
    pg = jnp.transpose(pg, (0, 1, 3, 2, 4, 5)).reshape(Bb, _n_G * _k, _n_G * _k, 3)
    pgi = jnp.transpose(pg, (0, 3, 1, 2)).reshape(Bb, 3, _N_G * _K)
    pgi = jnp.transpose(pgi, (0, 2, 1))
    return pgi


# speedup vs baseline: 8.1751x; 5.0964x over previous
import functools

import jax
import jax.numpy as jnp
from jax import lax
from jax.experimental import pallas as pl
from jax.experimental.pallas import tpu as pltpu
from jax.experimental.pallas import tpu_sc as plsc

_N_G = 256
_N_C = 64
_K = 16
_k = 4
_n_G = 16
_B = 8
_N = 8192


def _mlp_pallas(x, W1, b1, W2, b2, row_block=None):
    R, Din = x.shape
    H = W1.shape[1]
    Dout = W2.shape[1]
    if row_block is None:
        row_block = R

    def body(x_ref, w1_ref, b1_ref, w2_ref, b2_ref, o_ref):
        h = jnp.dot(x_ref[...], w1_ref[...], preferred_element_type=jnp.float32)
        h = jnp.maximum(h + b1_ref[...], 0.0)
        o_ref[...] = (
            jnp.dot(h, w2_ref[...], preferred_element_type=jnp.float32) + b2_ref[...]
        )

    return pl.pallas_call(
        body,
        grid=(R // row_block,),
        in_specs=[
            pl.BlockSpec((row_block, Din), lambda i: (i, 0)),
            pl.BlockSpec((Din, H), lambda i: (0, 0)),
            pl.BlockSpec((1, H), lambda i: (0, 0)),
            pl.BlockSpec((H, Dout), lambda i: (0, 0)),
            pl.BlockSpec((1, Dout), lambda i: (0, 0)),
        ],
        out_specs=pl.BlockSpec((row_block, Dout), lambda i: (i, 0)),
        out_shape=jax.ShapeDtypeStruct((R, Dout), jnp.float32),
    )(x, W1, b1.reshape(1, -1), W2, b2.reshape(1, -1))


def _fps_recg_pallas(pts, Wg1, bg1, Wg2, bg2):
    Bb = pts.shape[0]
    R = _N // 128
    ptsr = jnp.transpose(pts, (0, 2, 1)).reshape(Bb, 3, R, 128)

    def body(p_ref, w1_ref, b1_ref, w2_ref, b2_ref, g_ref, r_ref):
        px = p_ref[:, 0]
        py = p_ref[:, 1]
        pz = p_ref[:, 2]
        flat = (
            jax.lax.broadcasted_iota(jnp.int32, (Bb, R, 128), 1) * 128
            + jax.lax.broadcasted_iota(jnp.int32, (Bb, R, 128), 2)
        )

        def step(i, carry):
            dists, lx, ly, lz = carry
            d = (px - lx) ** 2 + (py - ly) ** 2 + (pz - lz) ** 2
            dists = jnp.minimum(dists, d)
            m = jnp.max(dists, axis=(1, 2), keepdims=True)
            cand = jnp.where(dists == m, flat, _N)
            nxt = jnp.min(cand, axis=(1, 2), keepdims=True)
            sel = flat == nxt
            nlx = jnp.sum(jnp.where(sel, px, 0.0), axis=(1, 2), keepdims=True)
            nly = jnp.sum(jnp.where(sel, py, 0.0), axis=(1, 2), keepdims=True)
            nlz = jnp.sum(jnp.where(sel, pz, 0.0), axis=(1, 2), keepdims=True)
            row = jnp.concatenate([nlx, nly, nlz], axis=2)
            g_ref[:, pl.ds(i, 1), :] = row
            return dists, nlx, nly, nlz

        lx0 = px[:, 0:1, 0:1]
        ly0 = py[:, 0:1, 0:1]
        lz0 = pz[:, 0:1, 0:1]
        g_ref[:, 0:1, :] = jnp.concatenate([lx0, ly0, lz0], axis=2)
        dists0 = jnp.full((Bb, R, 128), 1e10, jnp.float32)
        jax.lax.fori_loop(1, _N_G, step, (dists0, lx0, ly0, lz0), unroll=2)

        ptsg = g_ref[...].reshape(Bb * _N_G, 3)
        h = jnp.dot(ptsg, w1_ref[...], preferred_element_type=jnp.float32)
        h = jnp.maximum(h + b1_ref[...], 0.0)
        rec = jnp.dot(h, w2_ref[...], preferred_element_type=jnp.float32) + b2_ref[...]
        r_ref[...] = rec.reshape(Bb, _N_G, 3)

    return pl.pallas_call(
        body,
        out_shape=(
            jax.ShapeDtypeStruct((Bb, _N_G, 3), jnp.float32),
            jax.ShapeDtypeStruct((Bb, _N_G, 3), jnp.float32),
        ),
    )(ptsr, Wg1, bg1.reshape(1, -1), Wg2, bg2.reshape(1, -1))


def _knn_gather_sc(ptsT, rec_gT):
    NW = 32
    APW = (_B * _N_G) // NW
    NCH = _N // 16
    CAP = _N + 16
    mesh = plsc.VectorSubcoreMesh(core_axis_name="c", subcore_axis_name="s")

    @functools.partial(
        pl.kernel,
        out_type=jax.ShapeDtypeStruct((_B, _N_G, _N_C, 3), jnp.float32),
        mesh=mesh,
        compiler_params=pltpu.CompilerParams(
            use_tc_tiling_on_sc=False, needs_layout_passes=False
        ),
        scratch_types=[
            pltpu.VMEM((_N,), jnp.float32),
            pltpu.VMEM((_N,), jnp.float32),
            pltpu.VMEM((_N,), jnp.float32),
            pltpu.VMEM((_N,), jnp.float32),
            pltpu.VMEM((CAP,), jnp.int32),
            pltpu.VMEM((CAP,), jnp.int32),
            pltpu.VMEM((APW,), jnp.float32),
            pltpu.VMEM((APW,), jnp.float32),
            pltpu.VMEM((APW,), jnp.float32),
            pltpu.VMEM((_N_C + 16,), jnp.int32),
            pltpu.VMEM((APW, _N_C, 3), jnp.float32),
        ],
    )
    def knn(ptsT_hbm, recgT_hbm, out_hbm, px, py, pz, db, cd, ci, axr, ayr, azr, si, ob):
        cid = lax.axis_index("c")
        sid = lax.axis_index("s")
        w = cid * 16 + sid
        b = w // 4
        a0 = (w % 4) * APW
        pltpu.sync_copy(ptsT_hbm.at[b, 0], px)
        pltpu.sync_copy(ptsT_hbm.at[b, 1], py)
        pltpu.sync_copy(ptsT_hbm.at[b, 2], pz)
        pltpu.sync_copy(recgT_hbm.at[b, 0, pl.ds(a0, APW)], axr)
        pltpu.sync_copy(recgT_hbm.at[b, 1, pl.ds(a0, APW)], ayr)
        pltpu.sync_copy(recgT_hbm.at[b, 2, pl.ds(a0, APW)], azr)

        iota16 = lax.broadcasted_iota(jnp.int32, (16,), 0)
        inf16 = jnp.full((16,), jnp.inf, jnp.float32)
        K64 = _N_C

        def per_anchor(ai, _carry):
            base = (ai >> 4) << 4
            lane = ai & 15
            lsel = iota16 == lane
            axs = jnp.sum(jnp.where(lsel, axr[pl.ds(base, 16)], 0.0))
            ays = jnp.sum(jnp.where(lsel, ayr[pl.ds(base, 16)], 0.0))
            azs = jnp.sum(jnp.where(lsel, azr[pl.ds(base, 16)], 0.0))

            def p1(j, accs):
                a_0, a_1, a_2, a_3 = accs
                o = j * 16
                dx = px[pl.ds(o, 16)] - axs
                dy = py[pl.ds(o, 16)] - ays
                dz = pz[pl.ds(o, 16)] - azs
                dv = dx * dx + dy * dy + dz * dz
                db[pl.ds(o, 16)] = dv
                r = j & 3
                a_0 = jnp.where(r == 0, jnp.minimum(a_0, dv), a_0)
                a_1 = jnp.where(r == 1, jnp.minimum(a_1, dv), a_1)
                a_2 = jnp.where(r == 2, jnp.minimum(a_2, dv), a_2)
                a_3 = jnp.where(r == 3, jnp.minimum(a_3, dv), a_3)
                return a_0, a_1, a_2, a_3

            a_0, a_1, a_2, a_3 = lax.fori_loop(
                0, NCH, p1, (inf16, inf16, inf16, inf16)
            )
            T = jnp.max(jnp.maximum(jnp.maximum(a_0, a_1), jnp.maximum(a_2, a_3)))

            def p2(j, cnt):
                o = j * 16
                dv = db[pl.ds(o, 16)]
                m = dv <= T
                plsc.store_compressed(cd.at[pl.ds(cnt, 16)], plsc.bitcast(dv, jnp.int32), mask=m)
                plsc.store_compressed(ci.at[pl.ds(cnt, 16)], iota16 + o, mask=m)
                return cnt + jnp.sum(m.astype(jnp.int32))

            cnt = lax.fori_loop(0, NCH, p2, jnp.int32(0))
            cd[pl.ds(cnt, 16)] = jnp.full((16,), 0x7F800000, jnp.int32)
            nc = (cnt + 15) >> 4

            def count_le(mid):
                def cc(j, acc):
                    return acc + jnp.sum(
                        (cd[pl.ds(j * 16, 16)] <= mid).astype(jnp.int32)
                    )
                return lax.fori_loop(0, nc, cc, jnp.int32(0))

            def bs(_i, lohi):
                lo, hi = lohi
                mid = lo + ((hi - lo) >> 1)
                c = count_le(mid)
                return jnp.where(c >= K64, lo, mid + 1), jnp.where(c >= K64, mid, hi)

            lo, hi = lax.fori_loop(
                0, 31, bs, (jnp.int32(0), jnp.int32(0x7F800000))
            )
            v = hi
            c_lt = count_le(v - 1)
            need = K64 - c_lt

            def p3(j, carry):
                nsel, eqbase = carry
                cv = cd[pl.ds(j * 16, 16)]
                m_lt = cv < v
                m_eq = cv == v
                eqrank = plsc.cumsum(m_eq.astype(jnp.int32)) + eqbase
                m_sel = m_lt | (m_eq & (eqrank <= need))
                plsc.store_compressed(
                    si.at[pl.ds(nsel, 16)], ci[pl.ds(j * 16, 16)], mask=m_sel
                )
                return (
                    nsel + jnp.sum(m_sel.astype(jnp.int32)),
                    eqbase + jnp.sum(m_eq.astype(jnp.int32)),
                )

            lax.fori_loop(0, nc, p3, (jnp.int32(0), jnp.int32(0)))

            ia16 = jnp.full((16,), 0, jnp.int32) + ai
            for t in range(K64 // 16):
                iv = si[pl.ds(t * 16, 16)]
                ipt = iota16 + t * 16
                plsc.store_scatter(
                    ob, [ia16, ipt, jnp.zeros((16,), jnp.int32)],
                    plsc.load_gather(px, [iv]),
                )
                plsc.store_scatter(
                    ob, [ia16, ipt, jnp.full((16,), 1, jnp.int32)],
                    plsc.load_gather(py, [iv]),
                )
                plsc.store_scatter(
                    ob, [ia16, ipt, jnp.full((16,), 2, jnp.int32)],
                    plsc.load_gather(pz, [iv]),
                )
            return 0

        lax.fori_loop(0, APW, per_anchor, 0)
        pltpu.sync_copy(ob, out_hbm.at[b, pl.ds(a0, APW)])

    return knn(ptsT, rec_gT)


def kernel(pts, Wg1, bg1, Wg2, bg2, Wl1, bl1, Wl2, bl2):
    Bb = pts.shape[0]
    pts_g, rec_g = _fps_recg_pallas(pts, Wg1, bg1, Wg2, bg2)

    ptsT = jnp.transpose(pts, (0, 2, 1))
    rec_gT = jnp.transpose(rec_g, (0, 2, 1))
    pts_c = _knn_gather_sc(ptsT, rec_gT)

    center = jnp.mean(pts_c, axis=2, keepdims=True)
    centered = pts_c - center
    scale = jnp.max(
        jnp.linalg.norm(centered, axis=-1, keepdims=True), axis=2, keepdims=True
    ) + 1e-9
    pts_c_n = centered / scale

    e = _mlp_pallas(
        pts_c_n.reshape(Bb * _N_G * _N_C, 3), Wl1, bl1, Wl2, bl2, row_block=16384
    )
    e = e.reshape(Bb * _N_G, _N_C, 2)

    eps = 1e-6
    mn = jnp.min(e, axis=1, keepdims=True)
    mx = jnp.max(e, axis=1, keepdims=True)
    e = (e - mn) / (mx - mn + 1e-12) * (1.0 - 2.0 * eps) + eps
    ebd_c = e.reshape(Bb, _N_G, _N_C, 2)

    g = (jnp.arange(_k, dtype=jnp.float32) + 0.5) / _k
    gx, gy = jnp.meshgrid(g, g, indexing="ij")
    grid = jnp.stack([gx.reshape(-1), gy.reshape(-1)], axis=-1)
    gd = jnp.sum(
        (grid[None, None, :, None, :] - ebd_c[:, :, None, :, :]) ** 2, axis=-1
    )
    nn_idx = jnp.argmin(gd, axis=-1)
    idx3 = jnp.broadcast_to(nn_idx[..., None], nn_idx.shape + (3,))
    pgi_local = jnp.take_along_axis(pts_c, idx3, axis=2)

    pg = pgi_local.reshape(Bb, _n_G, _n_G, _k, _k, 3)
    pg = jnp.transpose(pg, (0, 1, 3, 2, 4, 5)).reshape(Bb, _n_G * _k, _n_G * _k, 3)
    pgi = jnp.transpose(pg, (0, 3, 1, 2)).reshape(Bb, 3, _N_G * _K)
    pgi = jnp.transpose(pgi, (0, 2, 1))
    return pgi


# speedup vs baseline: 9.7526x; 1.1930x over previous
import functools

import jax
import jax.numpy as jnp
from jax import lax
from jax.experimental import pallas as pl
from jax.experimental.pallas import tpu as pltpu
from jax.experimental.pallas import tpu_sc as plsc

_N_G = 256
_N_C = 64
_K = 16
_k = 4
_n_G = 16
_B = 8
_N = 8192


def _mlp_pallas(x, W1, b1, W2, b2, row_block=None):
    R, Din = x.shape
    H = W1.shape[1]
    Dout = W2.shape[1]
    if row_block is None:
        row_block = R

    def body(x_ref, w1_ref, b1_ref, w2_ref, b2_ref, o_ref):
        h = jnp.dot(x_ref[...], w1_ref[...], preferred_element_type=jnp.float32)
        h = jnp.maximum(h + b1_ref[...], 0.0)
        o_ref[...] = (
            jnp.dot(h, w2_ref[...], preferred_element_type=jnp.float32) + b2_ref[...]
        )

    return pl.pallas_call(
        body,
        grid=(R // row_block,),
        in_specs=[
            pl.BlockSpec((row_block, Din), lambda i: (i, 0)),
            pl.BlockSpec((Din, H), lambda i: (0, 0)),
            pl.BlockSpec((1, H), lambda i: (0, 0)),
            pl.BlockSpec((H, Dout), lambda i: (0, 0)),
            pl.BlockSpec((1, Dout), lambda i: (0, 0)),
        ],
        out_specs=pl.BlockSpec((row_block, Dout), lambda i: (i, 0)),
        out_shape=jax.ShapeDtypeStruct((R, Dout), jnp.float32),
    )(x, W1, b1.reshape(1, -1), W2, b2.reshape(1, -1))


def _fps_recg_pallas(pts, Wg1, bg1, Wg2, bg2):
    Bb = pts.shape[0]
    R = _N // 128
    ptsr = jnp.transpose(pts, (0, 2, 1)).reshape(Bb, 3, R, 128)

    def body(p_ref, w1_ref, b1_ref, w2_ref, b2_ref, g_ref, r_ref):
        px = p_ref[:, 0]
        py = p_ref[:, 1]
        pz = p_ref[:, 2]
        flat = (
            jax.lax.broadcasted_iota(jnp.int32, (Bb, R, 128), 1) * 128
            + jax.lax.broadcasted_iota(jnp.int32, (Bb, R, 128), 2)
        )

        def step(i, carry):
            dists, lx, ly, lz = carry
            d = (px - lx) ** 2 + (py - ly) ** 2 + (pz - lz) ** 2
            dists = jnp.minimum(dists, d)
            m = jnp.max(dists, axis=(1, 2), keepdims=True)
            cand = jnp.where(dists == m, flat, _N)
            nxt = jnp.min(cand, axis=(1, 2), keepdims=True)
            sel = flat == nxt
            nlx = jnp.sum(jnp.where(sel, px, 0.0), axis=(1, 2), keepdims=True)
            nly = jnp.sum(jnp.where(sel, py, 0.0), axis=(1, 2), keepdims=True)
            nlz = jnp.sum(jnp.where(sel, pz, 0.0), axis=(1, 2), keepdims=True)
            row = jnp.concatenate([nlx, nly, nlz], axis=2)
            g_ref[:, pl.ds(i, 1), :] = row
            return dists, nlx, nly, nlz

        lx0 = px[:, 0:1, 0:1]
        ly0 = py[:, 0:1, 0:1]
        lz0 = pz[:, 0:1, 0:1]
        g_ref[:, 0:1, :] = jnp.concatenate([lx0, ly0, lz0], axis=2)
        dists0 = jnp.full((Bb, R, 128), 1e10, jnp.float32)
        jax.lax.fori_loop(1, _N_G, step, (dists0, lx0, ly0, lz0), unroll=2)

        ptsg = g_ref[...].reshape(Bb * _N_G, 3)
        h = jnp.dot(ptsg, w1_ref[...], preferred_element_type=jnp.float32)
        h = jnp.maximum(h + b1_ref[...], 0.0)
        rec = jnp.dot(h, w2_ref[...], preferred_element_type=jnp.float32) + b2_ref[...]
        r_ref[...] = rec.reshape(Bb, _N_G, 3)

    return pl.pallas_call(
        body,
        out_shape=(
            jax.ShapeDtypeStruct((Bb, _N_G, 3), jnp.float32),
            jax.ShapeDtypeStruct((Bb, _N_G, 3), jnp.float32),
        ),
    )(ptsr, Wg1, bg1.reshape(1, -1), Wg2, bg2.reshape(1, -1))


def _knn_gather_sc(ptsT, rec_gT):
    NW = 32
    APW = (_B * _N_G) // NW
    NCH = _N // 16
    CAP = _N + 16
    mesh = plsc.VectorSubcoreMesh(core_axis_name="c", subcore_axis_name="s")

    @functools.partial(
        pl.kernel,
        out_type=jax.ShapeDtypeStruct((_B, _N_G, _N_C, 3), jnp.float32),
        mesh=mesh,
        compiler_params=pltpu.CompilerParams(
            use_tc_tiling_on_sc=False, needs_layout_passes=False
        ),
        scratch_types=[
            pltpu.VMEM((_N,), jnp.float32),
            pltpu.VMEM((_N,), jnp.float32),
            pltpu.VMEM((_N,), jnp.float32),
            pltpu.VMEM((_N,), jnp.float32),
            pltpu.VMEM((CAP,), jnp.int32),
            pltpu.VMEM((CAP,), jnp.int32),
            pltpu.VMEM((APW,), jnp.float32),
            pltpu.VMEM((APW,), jnp.float32),
            pltpu.VMEM((APW,), jnp.float32),
            pltpu.VMEM((_N_C + 16,), jnp.int32),
            pltpu.VMEM((APW, _N_C, 3), jnp.float32),
        ],
    )
    def knn(ptsT_hbm, recgT_hbm, out_hbm, px, py, pz, db, cd, ci, axr, ayr, azr, si, ob):
        cid = lax.axis_index("c")
        sid = lax.axis_index("s")
        w = cid * 16 + sid
        b = w // 4
        a0 = (w % 4) * APW
        pltpu.sync_copy(ptsT_hbm.at[b, 0], px)
        pltpu.sync_copy(ptsT_hbm.at[b, 1], py)
        pltpu.sync_copy(ptsT_hbm.at[b, 2], pz)
        pltpu.sync_copy(recgT_hbm.at[b, 0, pl.ds(a0, APW)], axr)
        pltpu.sync_copy(recgT_hbm.at[b, 1, pl.ds(a0, APW)], ayr)
        pltpu.sync_copy(recgT_hbm.at[b, 2, pl.ds(a0, APW)], azr)

        iota16 = lax.broadcasted_iota(jnp.int32, (16,), 0)
        inf16 = jnp.full((16,), jnp.inf, jnp.float32)
        K64 = _N_C

        def per_anchor(ai, _carry):
            base = (ai >> 4) << 4
            lane = ai & 15
            lsel = iota16 == lane
            axs = jnp.sum(jnp.where(lsel, axr[pl.ds(base, 16)], 0.0))
            ays = jnp.sum(jnp.where(lsel, ayr[pl.ds(base, 16)], 0.0))
            azs = jnp.sum(jnp.where(lsel, azr[pl.ds(base, 16)], 0.0))

            def p1(g, accs):
                o = g * 64
                new = []
                for r in range(4):
                    oo = o + r * 16
                    dx = px[pl.ds(oo, 16)] - axs
                    dy = py[pl.ds(oo, 16)] - ays
                    dz = pz[pl.ds(oo, 16)] - azs
                    dv = dx * dx + dy * dy + dz * dz
                    db[pl.ds(oo, 16)] = dv
                    new.append(jnp.minimum(accs[r], dv))
                return tuple(new)

            accs = lax.fori_loop(0, NCH // 4, p1, (inf16, inf16, inf16, inf16))
            maxv = jnp.maximum(jnp.maximum(accs[0], accs[1]),
                               jnp.maximum(accs[2], accs[3]))
            minv = jnp.minimum(jnp.minimum(accs[0], accs[1]),
                               jnp.minimum(accs[2], accs[3]))
            T = jnp.max(maxv)
            t_bits = jnp.max(plsc.bitcast(maxv, jnp.int32))
            lo_bits = jnp.min(plsc.bitcast(minv, jnp.int32))

            def p2(g, cnt):
                o = g * 64
                off = cnt
                for r in range(4):
                    oo = o + r * 16
                    dv = db[pl.ds(oo, 16)]
                    m = dv <= T
                    plsc.store_compressed(
                        cd.at[pl.ds(off, 16)], plsc.bitcast(dv, jnp.int32), mask=m
                    )
                    plsc.store_compressed(ci.at[pl.ds(off, 16)], iota16 + oo, mask=m)
                    off = off + jnp.sum(m.astype(jnp.int32))
                return off

            cnt = lax.fori_loop(0, NCH // 4, p2, jnp.int32(0))
            cd[pl.ds(cnt, 16)] = jnp.full((16,), 0x7F800000, jnp.int32)
            nc = (cnt + 15) >> 4

            one16 = jnp.full((16,), 1, jnp.int32)
            zero16 = jnp.zeros((16,), jnp.int32)

            def count_le(mid):
                def cc(j, acc):
                    return acc + jnp.where(cd[pl.ds(j * 16, 16)] <= mid, one16, zero16)
                return jnp.sum(lax.fori_loop(0, nc, cc, zero16))

            def bs_cond(lohi):
                return lohi[0] < lohi[1]

            def bs(lohi):
                lo, hi = lohi
                mid = lo + ((hi - lo) >> 1)
                c = count_le(mid)
                return jnp.where(c >= K64, lo, mid + 1), jnp.where(c >= K64, mid, hi)

            lo, hi = lax.while_loop(bs_cond, bs, (lo_bits, t_bits))
            v = hi
            c_lt = count_le(v - 1)
            need = K64 - c_lt

            def p3(j, carry):
                nsel, eqbase = carry
                cv = cd[pl.ds(j * 16, 16)]
                m_lt = cv < v
                m_eq = cv == v
                eqrank = plsc.cumsum(m_eq.astype(jnp.int32)) + eqbase
                m_sel = m_lt | (m_eq & (eqrank <= need))
                plsc.store_compressed(
                    si.at[pl.ds(nsel, 16)], ci[pl.ds(j * 16, 16)], mask=m_sel
                )
                return (
                    nsel + jnp.sum(m_sel.astype(jnp.int32)),
                    eqbase + jnp.sum(m_eq.astype(jnp.int32)),
                )

            lax.fori_loop(0, nc, p3, (jnp.int32(0), jnp.int32(0)))

            ia16 = jnp.full((16,), 0, jnp.int32) + ai
            for t in range(K64 // 16):
                iv = si[pl.ds(t * 16, 16)]
                ipt = iota16 + t * 16
                plsc.store_scatter(
                    ob, [ia16, ipt, jnp.zeros((16,), jnp.int32)],
                    plsc.load_gather(px, [iv]),
                )
                plsc.store_scatter(
                    ob, [ia16, ipt, jnp.full((16,), 1, jnp.int32)],
                    plsc.load_gather(py, [iv]),
                )
                plsc.store_scatter(
                    ob, [ia16, ipt, jnp.full((16,), 2, jnp.int32)],
                    plsc.load_gather(pz, [iv]),
                )
            return 0

        lax.fori_loop(0, APW, per_anchor, 0)
        pltpu.sync_copy(ob, out_hbm.at[b, pl.ds(a0, APW)])

    return knn(ptsT, rec_gT)


def kernel(pts, Wg1, bg1, Wg2, bg2, Wl1, bl1, Wl2, bl2):
    Bb = pts.shape[0]
    pts_g, rec_g = _fps_recg_pallas(pts, Wg1, bg1, Wg2, bg2)

    ptsT = jnp.transpose(pts, (0, 2, 1))
    rec_gT = jnp.transpose(rec_g, (0, 2, 1))
    pts_c = _knn_gather_sc(ptsT, rec_gT)

    center = jnp.mean(pts_c, axis=2, keepdims=True)
    centered = pts_c - center
    scale = jnp.max(
        jnp.linalg.norm(centered, axis=-1, keepdims=True), axis=2, keepdims=True
    ) + 1e-9
    pts_c_n = centered / scale

    e = _mlp_pallas(
        pts_c_n.reshape(Bb * _N_G * _N_C, 3), Wl1, bl1, Wl2, bl2, row_block=16384
    )
    e = e.reshape(Bb * _N_G, _N_C, 2)

    eps = 1e-6
    mn = jnp.min(e, axis=1, keepdims=True)
    mx = jnp.max(e, axis=1, keepdims=True)
    e = (e - mn) / (mx - mn + 1e-12) * (1.0 - 2.0 * eps) + eps
    ebd_c = e.reshape(Bb, _N_G, _N_C, 2)

    g = (jnp.arange(_k, dtype=jnp.float32) + 0.5) / _k
    gx, gy = jnp.meshgrid(g, g, indexing="ij")
    grid = jnp.stack([gx.reshape(-1), gy.reshape(-1)], axis=-1)
    gd = jnp.sum(
        (grid[None, None, :, None, :] - ebd_c[:, :, None, :, :]) ** 2, axis=-1
    )
    nn_idx = jnp.argmin(gd, axis=-1)
    idx3 = jnp.broadcast_to(nn_idx[..., None], nn_idx.shape + (3,))
    pgi_local = jnp.take_along_axis(pts_c, idx3, axis=2)

    pg = pgi_local.reshape(Bb, _n_G, _n_G, _k, _k, 3)
    pg = jnp.transpose(pg, (0, 1, 3, 2, 4, 5)).reshape(Bb, _n_G * _k, _n_G * _k, 3)
    pgi = jnp.transpose(pg, (0, 3, 1, 2)).reshape(Bb, 3, _N_G * _K)
    pgi = jnp.transpose(pgi, (0, 2, 1))
    return pgi


# speedup vs baseline: 10.9754x; 1.1254x over previous
import functools

import jax
import jax.numpy as jnp
from jax import lax
from jax.experimental import pallas as pl
from jax.experimental.pallas import tpu as pltpu
from jax.experimental.pallas import tpu_sc as plsc

_N_G = 256
_N_C = 64
_K = 16
_k = 4
_n_G = 16
_B = 8
_N = 8192


def _patch_embed_pallas(pts_c, Wl1, bl1, Wl2, bl2):
    P = _B * _N_G
    PB = 256
    rows = PB * _N_C

    def body(x_ref, w1_ref, b1_ref, w2_ref, b2_ref, o_ref):
        x3 = x_ref[...].reshape(PB, _N_C, 3)
        center = jnp.mean(x3, axis=1, keepdims=True)
        cent = x3 - center
        nrm = jnp.sqrt(jnp.sum(cent * cent, axis=2, keepdims=True))
        scale = jnp.max(nrm, axis=1, keepdims=True) + 1e-9
        xn = (cent / scale).reshape(rows, 3)
        h = jnp.dot(xn, w1_ref[...], preferred_element_type=jnp.float32)
        h = jnp.maximum(h + b1_ref[...], 0.0)
        e = jnp.dot(h, w2_ref[...], preferred_element_type=jnp.float32) + b2_ref[...]
        e3 = e.reshape(PB, _N_C, 2)
        mn = jnp.min(e3, axis=1, keepdims=True)
        mx = jnp.max(e3, axis=1, keepdims=True)
        eps = 1e-6
        e3 = (e3 - mn) / (mx - mn + 1e-12) * (1.0 - 2.0 * eps) + eps
        o_ref[...] = e3.reshape(rows, 2)

    return pl.pallas_call(
        body,
        grid=(P // PB,),
        in_specs=[
            pl.BlockSpec((rows, 3), lambda i: (i, 0)),
            pl.BlockSpec((3, 64), lambda i: (0, 0)),
            pl.BlockSpec((1, 64), lambda i: (0, 0)),
            pl.BlockSpec((64, 2), lambda i: (0, 0)),
            pl.BlockSpec((1, 2), lambda i: (0, 0)),
        ],
        out_specs=pl.BlockSpec((rows, 2), lambda i: (i, 0)),
        out_shape=jax.ShapeDtypeStruct((P * _N_C, 2), jnp.float32),
    )(pts_c.reshape(P * _N_C, 3), Wl1, bl1.reshape(1, -1), Wl2, bl2.reshape(1, -1))


def _resample_pallas(e2, pts_c):
    P = _B * _N_G
    PB = 256
    rows = PB * _N_C
    def body(e_ref, p_ref, o_ref):
        e3 = e_ref[...].reshape(PB, _N_C, 2)
        exv = e3[:, :, 0].reshape(PB, 1, _N_C)
        eyv = e3[:, :, 1].reshape(PB, 1, _N_C)
        iota_k = lax.broadcasted_iota(jnp.int32, (PB, _K, 1), 1)
        gx = ((iota_k // _k).astype(jnp.float32) + 0.5) / _k
        gy = ((iota_k % _k).astype(jnp.float32) + 0.5) / _k
        gd = (gx - exv) ** 2 + (gy - eyv) ** 2
        mn = jnp.min(gd, axis=2, keepdims=True)
        iota_c = lax.broadcasted_iota(jnp.int32, (PB, _K, _N_C), 2)
        sel = jnp.min(jnp.where(gd == mn, iota_c, _N_C), axis=2, keepdims=True)
        oh = (iota_c == sel).astype(jnp.float32)
        p3 = p_ref[...].reshape(PB, _N_C, 3)
        out = []
        for c in range(3):
            pc = p3[:, :, c].reshape(PB, 1, _N_C)
            out.append(jnp.sum(oh * pc, axis=2))
        o_ref[...] = jnp.stack(out, axis=-1)

    return pl.pallas_call(
        body,
        grid=(P // PB,),
        in_specs=[
            pl.BlockSpec((rows, 2), lambda i: (i, 0)),
            pl.BlockSpec((rows, 3), lambda i: (i, 0)),
        ],
        out_specs=pl.BlockSpec((PB, _K, 3), lambda i: (i, 0, 0)),
        out_shape=jax.ShapeDtypeStruct((P, _K, 3), jnp.float32),
    )(e2, pts_c.reshape(P * _N_C, 3))


def _mlp_pallas(x, W1, b1, W2, b2, row_block=None):
    R, Din = x.shape
    H = W1.shape[1]
    Dout = W2.shape[1]
    if row_block is None:
        row_block = R

    def body(x_ref, w1_ref, b1_ref, w2_ref, b2_ref, o_ref):
        h = jnp.dot(x_ref[...], w1_ref[...], preferred_element_type=jnp.float32)
        h = jnp.maximum(h + b1_ref[...], 0.0)
        o_ref[...] = (
            jnp.dot(h, w2_ref[...], preferred_element_type=jnp.float32) + b2_ref[...]
        )

    return pl.pallas_call(
        body,
        grid=(R // row_block,),
        in_specs=[
            pl.BlockSpec((row_block, Din), lambda i: (i, 0)),
            pl.BlockSpec((Din, H), lambda i: (0, 0)),
            pl.BlockSpec((1, H), lambda i: (0, 0)),
            pl.BlockSpec((H, Dout), lambda i: (0, 0)),
            pl.BlockSpec((1, Dout), lambda i: (0, 0)),
        ],
        out_specs=pl.BlockSpec((row_block, Dout), lambda i: (i, 0)),
        out_shape=jax.ShapeDtypeStruct((R, Dout), jnp.float32),
    )(x, W1, b1.reshape(1, -1), W2, b2.reshape(1, -1))


def _fps_recg_pallas(pts, Wg1, bg1, Wg2, bg2):
    Bb = pts.shape[0]
    R = _N // 128
    ptsr = jnp.transpose(pts, (0, 2, 1)).reshape(Bb, 3, R, 128)

    def body(p_ref, w1_ref, b1_ref, w2_ref, b2_ref, g_ref, r_ref):
        px = p_ref[:, 0]
        py = p_ref[:, 1]
        pz = p_ref[:, 2]
        flat = (
            jax.lax.broadcasted_iota(jnp.int32, (Bb, R, 128), 1) * 128
            + jax.lax.broadcasted_iota(jnp.int32, (Bb, R, 128), 2)
        )

        def step(i, carry):
            dists, lx, ly, lz = carry
            d = (px - lx) ** 2 + (py - ly) ** 2 + (pz - lz) ** 2
            dists = jnp.minimum(dists, d)
            m = jnp.max(dists, axis=(1, 2), keepdims=True)
            cand = jnp.where(dists == m, flat, _N)
            nxt = jnp.min(cand, axis=(1, 2), keepdims=True)
            sel = flat == nxt
            nlx = jnp.sum(jnp.where(sel, px, 0.0), axis=(1, 2), keepdims=True)
            nly = jnp.sum(jnp.where(sel, py, 0.0), axis=(1, 2), keepdims=True)
            nlz = jnp.sum(jnp.where(sel, pz, 0.0), axis=(1, 2), keepdims=True)
            row = jnp.concatenate([nlx, nly, nlz], axis=2)
            g_ref[:, pl.ds(i, 1), :] = row
            return dists, nlx, nly, nlz

        lx0 = px[:, 0:1, 0:1]
        ly0 = py[:, 0:1, 0:1]
        lz0 = pz[:, 0:1, 0:1]
        g_ref[:, 0:1, :] = jnp.concatenate([lx0, ly0, lz0], axis=2)
        dists0 = jnp.full((Bb, R, 128), 1e10, jnp.float32)
        jax.lax.fori_loop(1, _N_G, step, (dists0, lx0, ly0, lz0), unroll=2)

        ptsg = g_ref[...].reshape(Bb * _N_G, 3)
        h = jnp.dot(ptsg, w1_ref[...], preferred_element_type=jnp.float32)
        h = jnp.maximum(h + b1_ref[...], 0.0)
        rec = jnp.dot(h, w2_ref[...], preferred_element_type=jnp.float32) + b2_ref[...]
        r_ref[...] = rec.reshape(Bb, _N_G, 3)

    return pl.pallas_call(
        body,
        out_shape=(
            jax.ShapeDtypeStruct((Bb, _N_G, 3), jnp.float32),
            jax.ShapeDtypeStruct((Bb, _N_G, 3), jnp.float32),
        ),
    )(ptsr, Wg1, bg1.reshape(1, -1), Wg2, bg2.reshape(1, -1))


def _knn_gather_sc(ptsT, rec_gT):
    NW = 32
    APW = (_B * _N_G) // NW
    NCH = _N // 16
    CAP = _N + 16
    mesh = plsc.VectorSubcoreMesh(core_axis_name="c", subcore_axis_name="s")

    @functools.partial(
        pl.kernel,
        out_type=jax.ShapeDtypeStruct((_B, _N_G, _N_C, 3), jnp.float32),
        mesh=mesh,
        compiler_params=pltpu.CompilerParams(
            use_tc_tiling_on_sc=False, needs_layout_passes=False
        ),
        scratch_types=[
            pltpu.VMEM((_N,), jnp.float32),
            pltpu.VMEM((_N,), jnp.float32),
            pltpu.VMEM((_N,), jnp.float32),
            pltpu.VMEM((_N,), jnp.float32),
            pltpu.VMEM((CAP,), jnp.int32),
            pltpu.VMEM((CAP,), jnp.int32),
            pltpu.VMEM((APW,), jnp.float32),
            pltpu.VMEM((APW,), jnp.float32),
            pltpu.VMEM((APW,), jnp.float32),
            pltpu.VMEM((_N_C + 16,), jnp.int32),
            pltpu.VMEM((APW, _N_C, 3), jnp.float32),
        ],
    )
    def knn(ptsT_hbm, recgT_hbm, out_hbm, px, py, pz, db, cd, ci, axr, ayr, azr, si, ob):
        cid = lax.axis_index("c")
        sid = lax.axis_index("s")
        w = cid * 16 + sid
        b = w // 4
        a0 = (w % 4) * APW
        pltpu.sync_copy(ptsT_hbm.at[b, 0], px)
        pltpu.sync_copy(ptsT_hbm.at[b, 1], py)
        pltpu.sync_copy(ptsT_hbm.at[b, 2], pz)
        pltpu.sync_copy(recgT_hbm.at[b, 0, pl.ds(a0, APW)], axr)
        pltpu.sync_copy(recgT_hbm.at[b, 1, pl.ds(a0, APW)], ayr)
        pltpu.sync_copy(recgT_hbm.at[b, 2, pl.ds(a0, APW)], azr)

        iota16 = lax.broadcasted_iota(jnp.int32, (16,), 0)
        inf16 = jnp.full((16,), jnp.inf, jnp.float32)
        K64 = _N_C

        def per_anchor(ai, _carry):
            base = (ai >> 4) << 4
            lane = ai & 15
            lsel = iota16 == lane
            axs = jnp.sum(jnp.where(lsel, axr[pl.ds(base, 16)], 0.0))
            ays = jnp.sum(jnp.where(lsel, ayr[pl.ds(base, 16)], 0.0))
            azs = jnp.sum(jnp.where(lsel, azr[pl.ds(base, 16)], 0.0))

            def p1(g, accs):
                o = g * 64
                new = []
                for r in range(4):
                    oo = o + r * 16
                    dx = px[pl.ds(oo, 16)] - axs
                    dy = py[pl.ds(oo, 16)] - ays
                    dz = pz[pl.ds(oo, 16)] - azs
                    dv = dx * dx + dy * dy + dz * dz
                    db[pl.ds(oo, 16)] = dv
                    new.append(jnp.minimum(accs[r], dv))
                return tuple(new)

            accs = lax.fori_loop(0, NCH // 4, p1, (inf16, inf16, inf16, inf16))
            maxv = jnp.maximum(jnp.maximum(accs[0], accs[1]),
                               jnp.maximum(accs[2], accs[3]))
            minv = jnp.minimum(jnp.minimum(accs[0], accs[1]),
                               jnp.minimum(accs[2], accs[3]))
            T = jnp.max(maxv)
            t_bits = jnp.max(plsc.bitcast(maxv, jnp.int32))
            lo_bits = jnp.min(plsc.bitcast(minv, jnp.int32))

            def p2(g, cnt):
                o = g * 64
                off = cnt
                for r in range(4):
                    oo = o + r * 16
                    dv = db[pl.ds(oo, 16)]
                    m = dv <= T
                    plsc.store_compressed(
                        cd.at[pl.ds(off, 16)], plsc.bitcast(dv, jnp.int32), mask=m
                    )
                    plsc.store_compressed(ci.at[pl.ds(off, 16)], iota16 + oo, mask=m)
                    off = off + jnp.sum(m.astype(jnp.int32))
                return off

            cnt = lax.fori_loop(0, NCH // 4, p2, jnp.int32(0))
            cd[pl.ds(cnt, 16)] = jnp.full((16,), 0x7F800000, jnp.int32)
            nc = (cnt + 15) >> 4

            one16 = jnp.full((16,), 1, jnp.int32)
            zero16 = jnp.zeros((16,), jnp.int32)

            def count_le(mid):
                def cc(j, acc):
                    return acc + jnp.where(cd[pl.ds(j * 16, 16)] <= mid, one16, zero16)
                return jnp.sum(lax.fori_loop(0, nc, cc, zero16))

            def bs_cond(lohi):
                return lohi[0] < lohi[1]

            def bs(lohi):
                lo, hi = lohi
                mid = lo + ((hi - lo) >> 1)
                c = count_le(mid)
                return jnp.where(c >= K64, lo, mid + 1), jnp.where(c >= K64, mid, hi)

            lo, hi = lax.while_loop(bs_cond, bs, (lo_bits, t_bits))
            v = hi
            c_lt = count_le(v - 1)
            need = K64 - c_lt

            def p3(j, carry):
                nsel, eqbase = carry
                cv = cd[pl.ds(j * 16, 16)]
                m_lt = cv < v
                m_eq = cv == v
                eqrank = plsc.cumsum(m_eq.astype(jnp.int32)) + eqbase
                m_sel = m_lt | (m_eq & (eqrank <= need))
                plsc.store_compressed(
                    si.at[pl.ds(nsel, 16)], ci[pl.ds(j * 16, 16)], mask=m_sel
                )
                return (
                    nsel + jnp.sum(m_sel.astype(jnp.int32)),
                    eqbase + jnp.sum(m_eq.astype(jnp.int32)),
                )

            lax.fori_loop(0, nc, p3, (jnp.int32(0), jnp.int32(0)))

            ia16 = jnp.full((16,), 0, jnp.int32) + ai
            for t in range(K64 // 16):
                iv = si[pl.ds(t * 16, 16)]
                ipt = iota16 + t * 16
                plsc.store_scatter(
                    ob, [ia16, ipt, jnp.zeros((16,), jnp.int32)],
                    plsc.load_gather(px, [iv]),
                )
                plsc.store_scatter(
                    ob, [ia16, ipt, jnp.full((16,), 1, jnp.int32)],
                    plsc.load_gather(py, [iv]),
                )
                plsc.store_scatter(
                    ob, [ia16, ipt, jnp.full((16,), 2, jnp.int32)],
                    plsc.load_gather(pz, [iv]),
                )
            return 0

        lax.fori_loop(0, APW, per_anchor, 0)
        pltpu.sync_copy(ob, out_hbm.at[b, pl.ds(a0, APW)])

    return knn(ptsT, rec_gT)


def kernel(pts, Wg1, bg1, Wg2, bg2, Wl1, bl1, Wl2, bl2):
    Bb = pts.shape[0]
    pts_g, rec_g = _fps_recg_pallas(pts, Wg1, bg1, Wg2, bg2)

    ptsT = jnp.transpose(pts, (0, 2, 1))
    rec_gT = jnp.transpose(rec_g, (0, 2, 1))
    pts_c = _knn_gather_sc(ptsT, rec_gT)

    e2 = _patch_embed_pallas(pts_c, Wl1, bl1, Wl2, bl2)
    pgi_local = _resample_pallas(e2, pts_c)

    pg = pgi_local.reshape(Bb, _n_G, _n_G, _k, _k, 3)
    pg = jnp.transpose(pg, (0, 1, 3, 2, 4, 5)).reshape(Bb, _n_G * _k, _n_G * _k, 3)
    pgi = jnp.transpose(pg, (0, 3, 1, 2)).reshape(Bb, 3, _N_G * _K)
    pgi = jnp.transpose(pgi, (0, 2, 1))
    return pgi


# speedup vs baseline: 11.2018x; 1.0206x over previous
import functools

import jax
import jax.numpy as jnp
from jax import lax
from jax.experimental import pallas as pl
from jax.experimental.pallas import tpu as pltpu
from jax.experimental.pallas import tpu_sc as plsc

_N_G = 256
_N_C = 64
_K = 16
_k = 4
_n_G = 16
_B = 8
_N = 8192


def _patch_tail_pallas(pts_c, Wl1, bl1, Wl2, bl2):
    P = _B * _N_G
    PB = 256
    rows = PB * _N_C

    def body(x_ref, w1_ref, b1_ref, w2_ref, b2_ref, o_ref):
        x3 = x_ref[...].reshape(PB, _N_C, 3)
        center = jnp.mean(x3, axis=1, keepdims=True)
        cent = x3 - center
        nrm = jnp.sqrt(jnp.sum(cent * cent, axis=2, keepdims=True))
        scale = jnp.max(nrm, axis=1, keepdims=True) + 1e-9
        xn = (cent / scale).reshape(rows, 3)
        h = jnp.dot(xn, w1_ref[...], preferred_element_type=jnp.float32)
        h = jnp.maximum(h + b1_ref[...], 0.0)
        e = jnp.dot(h, w2_ref[...], preferred_element_type=jnp.float32) + b2_ref[...]
        e3 = e.reshape(PB, _N_C, 2)
        mn = jnp.min(e3, axis=1, keepdims=True)
        mx = jnp.max(e3, axis=1, keepdims=True)
        eps = 1e-6
        e3 = (e3 - mn) / (mx - mn + 1e-12) * (1.0 - 2.0 * eps) + eps

        exv = e3[:, :, 0].reshape(PB, 1, _N_C)
        eyv = e3[:, :, 1].reshape(PB, 1, _N_C)
        iota_k = lax.broadcasted_iota(jnp.int32, (PB, _K, 1), 1)
        gx = ((iota_k // _k).astype(jnp.float32) + 0.5) / _k
        gy = ((iota_k % _k).astype(jnp.float32) + 0.5) / _k
        gd = (gx - exv) ** 2 + (gy - eyv) ** 2
        gmn = jnp.min(gd, axis=2, keepdims=True)
        iota_c = lax.broadcasted_iota(jnp.int32, (PB, _K, _N_C), 2)
        sel = jnp.min(jnp.where(gd == gmn, iota_c, _N_C), axis=2, keepdims=True)
        oh = (iota_c == sel).astype(jnp.float32)
        out = []
        for c in range(3):
            pc = x3[:, :, c].reshape(PB, 1, _N_C)
            out.append(jnp.sum(oh * pc, axis=2))
        o_ref[...] = jnp.stack(out, axis=-1)

    return pl.pallas_call(
        body,
        grid=(P // PB,),
        in_specs=[
            pl.BlockSpec((rows, 3), lambda i: (i, 0)),
            pl.BlockSpec((3, 64), lambda i: (0, 0)),
            pl.BlockSpec((1, 64), lambda i: (0, 0)),
            pl.BlockSpec((64, 2), lambda i: (0, 0)),
            pl.BlockSpec((1, 2), lambda i: (0, 0)),
        ],
        out_specs=pl.BlockSpec((PB, _K, 3), lambda i: (i, 0, 0)),
        out_shape=jax.ShapeDtypeStruct((P, _K, 3), jnp.float32),
    )(pts_c.reshape(P * _N_C, 3), Wl1, bl1.reshape(1, -1), Wl2, bl2.reshape(1, -1))


def _mlp_pallas(x, W1, b1, W2, b2, row_block=None):
    R, Din = x.shape
    H = W1.shape[1]
    Dout = W2.shape[1]
    if row_block is None:
        row_block = R

    def body(x_ref, w1_ref, b1_ref, w2_ref, b2_ref, o_ref):
        h = jnp.dot(x_ref[...], w1_ref[...], preferred_element_type=jnp.float32)
        h = jnp.maximum(h + b1_ref[...], 0.0)
        o_ref[...] = (
            jnp.dot(h, w2_ref[...], preferred_element_type=jnp.float32) + b2_ref[...]
        )

    return pl.pallas_call(
        body,
        grid=(R // row_block,),
        in_specs=[
            pl.BlockSpec((row_block, Din), lambda i: (i, 0)),
            pl.BlockSpec((Din, H), lambda i: (0, 0)),
            pl.BlockSpec((1, H), lambda i: (0, 0)),
            pl.BlockSpec((H, Dout), lambda i: (0, 0)),
            pl.BlockSpec((1, Dout), lambda i: (0, 0)),
        ],
        out_specs=pl.BlockSpec((row_block, Dout), lambda i: (i, 0)),
        out_shape=jax.ShapeDtypeStruct((R, Dout), jnp.float32),
    )(x, W1, b1.reshape(1, -1), W2, b2.reshape(1, -1))


def _fps_recg_pallas(pts, Wg1, bg1, Wg2, bg2):
    Bb = pts.shape[0]
    R = _N // 128
    ptsr = jnp.transpose(pts, (0, 2, 1)).reshape(Bb, 3, R, 128)

    def body(p_ref, w1_ref, b1_ref, w2_ref, b2_ref, g_ref, r_ref):
        px = p_ref[:, 0]
        py = p_ref[:, 1]
        pz = p_ref[:, 2]
        flat = (
            jax.lax.broadcasted_iota(jnp.int32, (Bb, R, 128), 1) * 128
            + jax.lax.broadcasted_iota(jnp.int32, (Bb, R, 128), 2)
        )

        def step(i, carry):
            dists, lx, ly, lz = carry
            d = (px - lx) ** 2 + (py - ly) ** 2 + (pz - lz) ** 2
            dists = jnp.minimum(dists, d)
            m = jnp.max(dists, axis=(1, 2), keepdims=True)
            cand = jnp.where(dists == m, flat, _N)
            nxt = jnp.min(cand, axis=(1, 2), keepdims=True)
            sel = flat == nxt
            nlx = jnp.sum(jnp.where(sel, px, 0.0), axis=(1, 2), keepdims=True)
            nly = jnp.sum(jnp.where(sel, py, 0.0), axis=(1, 2), keepdims=True)
            nlz = jnp.sum(jnp.where(sel, pz, 0.0), axis=(1, 2), keepdims=True)
            row = jnp.concatenate([nlx, nly, nlz], axis=2)
            g_ref[:, pl.ds(i, 1), :] = row
            return dists, nlx, nly, nlz

        lx0 = px[:, 0:1, 0:1]
        ly0 = py[:, 0:1, 0:1]
        lz0 = pz[:, 0:1, 0:1]
        g_ref[:, 0:1, :] = jnp.concatenate([lx0, ly0, lz0], axis=2)
        dists0 = jnp.full((Bb, R, 128), 1e10, jnp.float32)
        jax.lax.fori_loop(1, _N_G, step, (dists0, lx0, ly0, lz0), unroll=2)

        ptsg = g_ref[...].reshape(Bb * _N_G, 3)
        h = jnp.dot(ptsg, w1_ref[...], preferred_element_type=jnp.float32)
        h = jnp.maximum(h + b1_ref[...], 0.0)
        rec = jnp.dot(h, w2_ref[...], preferred_element_type=jnp.float32) + b2_ref[...]
        r_ref[...] = rec.reshape(Bb, _N_G, 3)

    return pl.pallas_call(
        body,
        out_shape=(
            jax.ShapeDtypeStruct((Bb, _N_G, 3), jnp.float32),
            jax.ShapeDtypeStruct((Bb, _N_G, 3), jnp.float32),
        ),
    )(ptsr, Wg1, bg1.reshape(1, -1), Wg2, bg2.reshape(1, -1))


def _knn_gather_sc(ptsT, rec_gT):
    NW = 32
    APW = (_B * _N_G) // NW
    NCH = _N // 16
    CAP = _N + 16
    mesh = plsc.VectorSubcoreMesh(core_axis_name="c", subcore_axis_name="s")

    @functools.partial(
        pl.kernel,
        out_type=jax.ShapeDtypeStruct((_B, _N_G, _N_C, 3), jnp.float32),
        mesh=mesh,
        compiler_params=pltpu.CompilerParams(
            use_tc_tiling_on_sc=False, needs_layout_passes=False
        ),
        scratch_types=[
            pltpu.VMEM((_N,), jnp.float32),
            pltpu.VMEM((_N,), jnp.float32),
            pltpu.VMEM((_N,), jnp.float32),
            pltpu.VMEM((_N,), jnp.float32),
            pltpu.VMEM((CAP,), jnp.int32),
            pltpu.VMEM((CAP,), jnp.int32),
            pltpu.VMEM((APW,), jnp.float32),
            pltpu.VMEM((APW,), jnp.float32),
            pltpu.VMEM((APW,), jnp.float32),
            pltpu.VMEM((_N_C + 16,), jnp.int32),
            pltpu.VMEM((APW, _N_C, 3), jnp.float32),
        ],
    )
    def knn(ptsT_hbm, recgT_hbm, out_hbm, px, py, pz, db, cd, ci, axr, ayr, azr, si, ob):
        cid = lax.axis_index("c")
        sid = lax.axis_index("s")
        w = cid * 16 + sid
        b = w // 4
        a0 = (w % 4) * APW
        pltpu.sync_copy(ptsT_hbm.at[b, 0], px)
        pltpu.sync_copy(ptsT_hbm.at[b, 1], py)
        pltpu.sync_copy(ptsT_hbm.at[b, 2], pz)
        pltpu.sync_copy(recgT_hbm.at[b, 0, pl.ds(a0, APW)], axr)
        pltpu.sync_copy(recgT_hbm.at[b, 1, pl.ds(a0, APW)], ayr)
        pltpu.sync_copy(recgT_hbm.at[b, 2, pl.ds(a0, APW)], azr)

        iota16 = lax.broadcasted_iota(jnp.int32, (16,), 0)
        inf16 = jnp.full((16,), jnp.inf, jnp.float32)
        K64 = _N_C

        def per_anchor(ai, _carry):
            base = (ai >> 4) << 4
            lane = ai & 15
            lsel = iota16 == lane
            axs = jnp.sum(jnp.where(lsel, axr[pl.ds(base, 16)], 0.0))
            ays = jnp.sum(jnp.where(lsel, ayr[pl.ds(base, 16)], 0.0))
            azs = jnp.sum(jnp.where(lsel, azr[pl.ds(base, 16)], 0.0))

            def p1(g, accs):
                o = g * 128
                new = list(accs)
                for r in range(8):
                    oo = o + r * 16
                    dx = px[pl.ds(oo, 16)] - axs
                    dy = py[pl.ds(oo, 16)] - ays
                    dz = pz[pl.ds(oo, 16)] - azs
                    dv = dx * dx + dy * dy + dz * dz
                    db[pl.ds(oo, 16)] = dv
                    new[r & 3] = jnp.minimum(new[r & 3], dv)
                return tuple(new)

            accs = lax.fori_loop(0, NCH // 8, p1, (inf16, inf16, inf16, inf16))
            maxv = jnp.maximum(jnp.maximum(accs[0], accs[1]),
                               jnp.maximum(accs[2], accs[3]))
            minv = jnp.minimum(jnp.minimum(accs[0], accs[1]),
                               jnp.minimum(accs[2], accs[3]))
            T = jnp.max(maxv)
            t_bits = jnp.max(plsc.bitcast(maxv, jnp.int32))
            lo_bits = jnp.min(plsc.bitcast(minv, jnp.int32))

            def p2(g, cnt):
                o = g * 64
                off = cnt
                for r in range(4):
                    oo = o + r * 16
                    dv = db[pl.ds(oo, 16)]
                    m = dv <= T
                    plsc.store_compressed(
                        cd.at[pl.ds(off, 16)], plsc.bitcast(dv, jnp.int32), mask=m
                    )
                    plsc.store_compressed(ci.at[pl.ds(off, 16)], iota16 + oo, mask=m)
                    off = off + jnp.sum(m.astype(jnp.int32))
                return off

            cnt = lax.fori_loop(0, NCH // 4, p2, jnp.int32(0))
            cd[pl.ds(cnt, 16)] = jnp.full((16,), 0x7F800000, jnp.int32)
            nc = (cnt + 15) >> 4

            one16 = jnp.full((16,), 1, jnp.int32)
            zero16 = jnp.zeros((16,), jnp.int32)

            def count_le(mid):
                def cc(j, acc):
                    return acc + jnp.where(cd[pl.ds(j * 16, 16)] <= mid, one16, zero16)
                return jnp.sum(lax.fori_loop(0, nc, cc, zero16))

            def bs_cond(lohi):
                return lohi[0] < lohi[1]

            def bs(lohi):
                lo, hi = lohi
                mid = lo + ((hi - lo) >> 1)
                c = count_le(mid)
                return jnp.where(c >= K64, lo, mid + 1), jnp.where(c >= K64, mid, hi)

            lo, hi = lax.while_loop(bs_cond, bs, (lo_bits, t_bits))
            v = hi
            c_lt = count_le(v - 1)
            need = K64 - c_lt

            def p3(j, carry):
                nsel, eqbase = carry
                cv = cd[pl.ds(j * 16, 16)]
                m_lt = cv < v
                m_eq = cv == v
                eqrank = plsc.cumsum(m_eq.astype(jnp.int32)) + eqbase
                m_sel = m_lt | (m_eq & (eqrank <= need))
                plsc.store_compressed(
                    si.at[pl.ds(nsel, 16)], ci[pl.ds(j * 16, 16)], mask=m_sel
                )
                return (
                    nsel + jnp.sum(m_sel.astype(jnp.int32)),
                    eqbase + jnp.sum(m_eq.astype(jnp.int32)),
                )

            lax.fori_loop(0, nc, p3, (jnp.int32(0), jnp.int32(0)))

            ia16 = jnp.full((16,), 0, jnp.int32) + ai
            for t in range(K64 // 16):
                iv = si[pl.ds(t * 16, 16)]
                ipt = iota16 + t * 16
                plsc.store_scatter(
                    ob, [ia16, ipt, jnp.zeros((16,), jnp.int32)],
                    plsc.load_gather(px, [iv]),
                )
                plsc.store_scatter(
                    ob, [ia16, ipt, jnp.full((16,), 1, jnp.int32)],
                    plsc.load_gather(py, [iv]),
                )
                plsc.store_scatter(
                    ob, [ia16, ipt, jnp.full((16,), 2, jnp.int32)],
                    plsc.load_gather(pz, [iv]),
                )
            return 0

        lax.fori_loop(0, APW, per_anchor, 0)
        pltpu.sync_copy(ob, out_hbm.at[b, pl.ds(a0, APW)])

    return knn(ptsT, rec_gT)


def kernel(pts, Wg1, bg1, Wg2, bg2, Wl1, bl1, Wl2, bl2):
    Bb = pts.shape[0]
    pts_g, rec_g = _fps_recg_pallas(pts, Wg1, bg1, Wg2, bg2)

    ptsT = jnp.transpose(pts, (0, 2, 1))
    rec_gT = jnp.transpose(rec_g, (0, 2, 1))
    pts_c = _knn_gather_sc(ptsT, rec_gT)

    pgi_local = _patch_tail_pallas(pts_c, Wl1, bl1, Wl2, bl2)

    pg = pgi_local.reshape(Bb, _n_G, _n_G, _k, _k, 3)
    pg = jnp.transpose(pg, (0, 1, 3, 2, 4, 5)).reshape(Bb, _n_G * _k, _n_G * _k, 3)
    pgi = jnp.transpose(pg, (0, 3, 1, 2)).reshape(Bb, 3, _N_G * _K)
    pgi = jnp.transpose(pgi, (0, 2, 1))
    return pgi


# speedup vs baseline: 12.5311x; 1.1187x over previous
import functools

import jax
import jax.numpy as jnp
from jax import lax
from jax.experimental import pallas as pl
from jax.experimental.pallas import tpu as pltpu
from jax.experimental.pallas import tpu_sc as plsc

_N_G = 256
_N_C = 64
_K = 16
_k = 4
_n_G = 16
_B = 8
_N = 8192


def _patch_tail_pallas(pts_c, Wl1, bl1, Wl2, bl2):
    P = pts_c.shape[0] * _N_G
    PB = 256
    rows = PB * _N_C

    def body(x_ref, w1_ref, b1_ref, w2_ref, b2_ref, o_ref):
        x3 = x_ref[...].reshape(PB, _N_C, 3)
        center = jnp.mean(x3, axis=1, keepdims=True)
        cent = x3 - center
        nrm = jnp.sqrt(jnp.sum(cent * cent, axis=2, keepdims=True))
        scale = jnp.max(nrm, axis=1, keepdims=True) + 1e-9
        xn = (cent / scale).reshape(rows, 3)
        h = jnp.dot(xn, w1_ref[...], preferred_element_type=jnp.float32)
        h = jnp.maximum(h + b1_ref[...], 0.0)
        e = jnp.dot(h, w2_ref[...], preferred_element_type=jnp.float32) + b2_ref[...]
        e3 = e.reshape(PB, _N_C, 2)
        mn = jnp.min(e3, axis=1, keepdims=True)
        mx = jnp.max(e3, axis=1, keepdims=True)
        eps = 1e-6
        e3 = (e3 - mn) / (mx - mn + 1e-12) * (1.0 - 2.0 * eps) + eps

        exv = e3[:, :, 0].reshape(PB, 1, _N_C)
        eyv = e3[:, :, 1].reshape(PB, 1, _N_C)
        iota_k = lax.broadcasted_iota(jnp.int32, (PB, _K, 1), 1)
        gx = ((iota_k // _k).astype(jnp.float32) + 0.5) / _k
        gy = ((iota_k % _k).astype(jnp.float32) + 0.5) / _k
        gd = (gx - exv) ** 2 + (gy - eyv) ** 2
        gmn = jnp.min(gd, axis=2, keepdims=True)
        iota_c = lax.broadcasted_iota(jnp.int32, (PB, _K, _N_C), 2)
        sel = jnp.min(jnp.where(gd == gmn, iota_c, _N_C), axis=2, keepdims=True)
        oh = (iota_c == sel).astype(jnp.float32)
        out = []
        for c in range(3):
            pc = x3[:, :, c].reshape(PB, 1, _N_C)
            out.append(jnp.sum(oh * pc, axis=2))
        o_ref[...] = jnp.stack(out, axis=-1)

    return pl.pallas_call(
        body,
        grid=(P // PB,),
        in_specs=[
            pl.BlockSpec((rows, 3), lambda i: (i, 0)),
            pl.BlockSpec((3, 64), lambda i: (0, 0)),
            pl.BlockSpec((1, 64), lambda i: (0, 0)),
            pl.BlockSpec((64, 2), lambda i: (0, 0)),
            pl.BlockSpec((1, 2), lambda i: (0, 0)),
        ],
        out_specs=pl.BlockSpec((PB, _K, 3), lambda i: (i, 0, 0)),
        out_shape=jax.ShapeDtypeStruct((P, _K, 3), jnp.float32),
    )(pts_c.reshape(P * _N_C, 3), Wl1, bl1.reshape(1, -1), Wl2, bl2.reshape(1, -1))


def _mlp_pallas(x, W1, b1, W2, b2, row_block=None):
    R, Din = x.shape
    H = W1.shape[1]
    Dout = W2.shape[1]
    if row_block is None:
        row_block = R

    def body(x_ref, w1_ref, b1_ref, w2_ref, b2_ref, o_ref):
        h = jnp.dot(x_ref[...], w1_ref[...], preferred_element_type=jnp.float32)
        h = jnp.maximum(h + b1_ref[...], 0.0)
        o_ref[...] = (
            jnp.dot(h, w2_ref[...], preferred_element_type=jnp.float32) + b2_ref[...]
        )

    return pl.pallas_call(
        body,
        grid=(R // row_block,),
        in_specs=[
            pl.BlockSpec((row_block, Din), lambda i: (i, 0)),
            pl.BlockSpec((Din, H), lambda i: (0, 0)),
            pl.BlockSpec((1, H), lambda i: (0, 0)),
            pl.BlockSpec((H, Dout), lambda i: (0, 0)),
            pl.BlockSpec((1, Dout), lambda i: (0, 0)),
        ],
        out_specs=pl.BlockSpec((row_block, Dout), lambda i: (i, 0)),
        out_shape=jax.ShapeDtypeStruct((R, Dout), jnp.float32),
    )(x, W1, b1.reshape(1, -1), W2, b2.reshape(1, -1))


def _fps_recg_pallas(pts, Wg1, bg1, Wg2, bg2):
    Bb = pts.shape[0]
    R = _N // 128
    ptsr = jnp.transpose(pts, (0, 2, 1)).reshape(Bb, 3, R, 128)

    def body(p_ref, w1_ref, b1_ref, w2_ref, b2_ref, g_ref, r_ref):
        px = p_ref[:, 0]
        py = p_ref[:, 1]
        pz = p_ref[:, 2]
        flat = (
            jax.lax.broadcasted_iota(jnp.int32, (Bb, R, 128), 1) * 128
            + jax.lax.broadcasted_iota(jnp.int32, (Bb, R, 128), 2)
        )

        def step(i, carry):
            dists, lx, ly, lz = carry
            d = (px - lx) ** 2 + (py - ly) ** 2 + (pz - lz) ** 2
            dists = jnp.minimum(dists, d)
            m = jnp.max(dists, axis=(1, 2), keepdims=True)
            cand = jnp.where(dists == m, flat, _N)
            nxt = jnp.min(cand, axis=(1, 2), keepdims=True)
            sel = flat == nxt
            nlx = jnp.sum(jnp.where(sel, px, 0.0), axis=(1, 2), keepdims=True)
            nly = jnp.sum(jnp.where(sel, py, 0.0), axis=(1, 2), keepdims=True)
            nlz = jnp.sum(jnp.where(sel, pz, 0.0), axis=(1, 2), keepdims=True)
            row = jnp.concatenate([nlx, nly, nlz], axis=2)
            g_ref[:, pl.ds(i, 1), :] = row
            return dists, nlx, nly, nlz

        lx0 = px[:, 0:1, 0:1]
        ly0 = py[:, 0:1, 0:1]
        lz0 = pz[:, 0:1, 0:1]
        g_ref[:, 0:1, :] = jnp.concatenate([lx0, ly0, lz0], axis=2)
        dists0 = jnp.full((Bb, R, 128), 1e10, jnp.float32)
        jax.lax.fori_loop(1, _N_G, step, (dists0, lx0, ly0, lz0), unroll=2)

        ptsg = g_ref[...].reshape(Bb * _N_G, 3)
        h = jnp.dot(ptsg, w1_ref[...], preferred_element_type=jnp.float32)
        h = jnp.maximum(h + b1_ref[...], 0.0)
        rec = jnp.dot(h, w2_ref[...], preferred_element_type=jnp.float32) + b2_ref[...]
        r_ref[...] = rec.reshape(Bb, _N_G, 3)

    return pl.pallas_call(
        body,
        out_shape=(
            jax.ShapeDtypeStruct((Bb, _N_G, 3), jnp.float32),
            jax.ShapeDtypeStruct((Bb, _N_G, 3), jnp.float32),
        ),
    )(ptsr, Wg1, bg1.reshape(1, -1), Wg2, bg2.reshape(1, -1))


def _knn_gather_sc(ptsT, rec_gT):
    NW = 32
    Bh = ptsT.shape[0]
    WPB = NW // Bh
    APW = _N_G // WPB
    NCH = _N // 16
    CAP = _N + 16
    mesh = plsc.VectorSubcoreMesh(core_axis_name="c", subcore_axis_name="s")

    @functools.partial(
        pl.kernel,
        out_type=jax.ShapeDtypeStruct((Bh, _N_G, _N_C, 3), jnp.float32),
        mesh=mesh,
        compiler_params=pltpu.CompilerParams(
            use_tc_tiling_on_sc=False, needs_layout_passes=False
        ),
        scratch_types=[
            pltpu.VMEM((_N,), jnp.float32),
            pltpu.VMEM((_N,), jnp.float32),
            pltpu.VMEM((_N,), jnp.float32),
            pltpu.VMEM((_N,), jnp.float32),
            pltpu.VMEM((CAP,), jnp.int32),
            pltpu.VMEM((CAP,), jnp.int32),
            pltpu.VMEM((APW,), jnp.float32),
            pltpu.VMEM((APW,), jnp.float32),
            pltpu.VMEM((APW,), jnp.float32),
            pltpu.VMEM((_N_C + 16,), jnp.int32),
            pltpu.VMEM((APW, _N_C, 3), jnp.float32),
        ],
    )
    def knn(ptsT_hbm, recgT_hbm, out_hbm, px, py, pz, db, cd, ci, axr, ayr, azr, si, ob):
        cid = lax.axis_index("c")
        sid = lax.axis_index("s")
        w = cid * 16 + sid
        b = w // WPB
        a0 = (w % WPB) * APW
        pltpu.sync_copy(ptsT_hbm.at[b, 0], px)
        pltpu.sync_copy(ptsT_hbm.at[b, 1], py)
        pltpu.sync_copy(ptsT_hbm.at[b, 2], pz)
        pltpu.sync_copy(recgT_hbm.at[b, 0, pl.ds(a0, APW)], axr)
        pltpu.sync_copy(recgT_hbm.at[b, 1, pl.ds(a0, APW)], ayr)
        pltpu.sync_copy(recgT_hbm.at[b, 2, pl.ds(a0, APW)], azr)

        iota16 = lax.broadcasted_iota(jnp.int32, (16,), 0)
        inf16 = jnp.full((16,), jnp.inf, jnp.float32)
        K64 = _N_C

        def per_anchor(ai, _carry):
            base = (ai >> 4) << 4
            lane = ai & 15
            lsel = iota16 == lane
            axs = jnp.sum(jnp.where(lsel, axr[pl.ds(base, 16)], 0.0))
            ays = jnp.sum(jnp.where(lsel, ayr[pl.ds(base, 16)], 0.0))
            azs = jnp.sum(jnp.where(lsel, azr[pl.ds(base, 16)], 0.0))

            def p1(g, accs):
                o = g * 128
                new = list(accs)
                for r in range(8):
                    oo = o + r * 16
                    dx = px[pl.ds(oo, 16)] - axs
                    dy = py[pl.ds(oo, 16)] - ays
                    dz = pz[pl.ds(oo, 16)] - azs
                    dv = dx * dx + dy * dy + dz * dz
                    db[pl.ds(oo, 16)] = dv
                    new[r & 3] = jnp.minimum(new[r & 3], dv)
                return tuple(new)

            accs = lax.fori_loop(0, NCH // 8, p1, (inf16, inf16, inf16, inf16))
            maxv = jnp.maximum(jnp.maximum(accs[0], accs[1]),
                               jnp.maximum(accs[2], accs[3]))
            minv = jnp.minimum(jnp.minimum(accs[0], accs[1]),
                               jnp.minimum(accs[2], accs[3]))
            T = jnp.max(maxv)
            t_bits = jnp.max(plsc.bitcast(maxv, jnp.int32))
            lo_bits = jnp.min(plsc.bitcast(minv, jnp.int32))

            def p2(g, cnt):
                o = g * 64
                off = cnt
                for r in range(4):
                    oo = o + r * 16
                    dv = db[pl.ds(oo, 16)]
                    m = dv <= T
                    plsc.store_compressed(
                        cd.at[pl.ds(off, 16)], plsc.bitcast(dv, jnp.int32), mask=m
                    )
                    plsc.store_compressed(ci.at[pl.ds(off, 16)], iota16 + oo, mask=m)
                    off = off + jnp.sum(m.astype(jnp.int32))
                return off

            cnt = lax.fori_loop(0, NCH // 4, p2, jnp.int32(0))
            cd[pl.ds(cnt, 16)] = jnp.full((16,), 0x7F800000, jnp.int32)
            nc = (cnt + 15) >> 4

            one16 = jnp.full((16,), 1, jnp.int32)
            zero16 = jnp.zeros((16,), jnp.int32)

            def count_le(mid):
                def cc(j, acc):
                    return acc + jnp.where(cd[pl.ds(j * 16, 16)] <= mid, one16, zero16)
                return jnp.sum(lax.fori_loop(0, nc, cc, zero16))

            def bs_cond(lohi):
                return lohi[0] < lohi[1]

            def bs(lohi):
                lo, hi = lohi
                mid = lo + ((hi - lo) >> 1)
                c = count_le(mid)
                return jnp.where(c >= K64, lo, mid + 1), jnp.where(c >= K64, mid, hi)

            lo, hi = lax.while_loop(bs_cond, bs, (lo_bits, t_bits))
            v = hi
            c_lt = count_le(v - 1)
            need = K64 - c_lt

            def p3(j, carry):
                nsel, eqbase = carry
                cv = cd[pl.ds(j * 16, 16)]
                m_lt = cv < v
                m_eq = cv == v
                eqrank = plsc.cumsum(m_eq.astype(jnp.int32)) + eqbase
                m_sel = m_lt | (m_eq & (eqrank <= need))
                plsc.store_compressed(
                    si.at[pl.ds(nsel, 16)], ci[pl.ds(j * 16, 16)], mask=m_sel
                )
                return (
                    nsel + jnp.sum(m_sel.astype(jnp.int32)),
                    eqbase + jnp.sum(m_eq.astype(jnp.int32)),
                )

            lax.fori_loop(0, nc, p3, (jnp.int32(0), jnp.int32(0)))

            ia16 = jnp.full((16,), 0, jnp.int32) + ai
            for t in range(K64 // 16):
                iv = si[pl.ds(t * 16, 16)]
                ipt = iota16 + t * 16
                plsc.store_scatter(
                    ob, [ia16, ipt, jnp.zeros((16,), jnp.int32)],
                    plsc.load_gather(px, [iv]),
                )
                plsc.store_scatter(
                    ob, [ia16, ipt, jnp.full((16,), 1, jnp.int32)],
                    plsc.load_gather(py, [iv]),
                )
                plsc.store_scatter(
                    ob, [ia16, ipt, jnp.full((16,), 2, jnp.int32)],
                    plsc.load_gather(pz, [iv]),
                )
            return 0

        lax.fori_loop(0, APW, per_anchor, 0)
        pltpu.sync_copy(ob, out_hbm.at[b, pl.ds(a0, APW)])

    return knn(ptsT, rec_gT)


def kernel(pts, Wg1, bg1, Wg2, bg2, Wl1, bl1, Wl2, bl2):
    Bb = pts.shape[0]
    Bh = Bb // 2
    halves = []
    for h in range(2):
        ph = pts[h * Bh:(h + 1) * Bh]
        pts_g, rec_g = _fps_recg_pallas(ph, Wg1, bg1, Wg2, bg2)
        ptsT = jnp.transpose(ph, (0, 2, 1))
        rec_gT = jnp.transpose(rec_g, (0, 2, 1))
        pts_c = _knn_gather_sc(ptsT, rec_gT)
        halves.append(_patch_tail_pallas(pts_c, Wl1, bl1, Wl2, bl2))

    pgi_local = jnp.concatenate(halves, axis=0)

    pg = pgi_local.reshape(Bb, _n_G, _n_G, _k, _k, 3)
    pg = jnp.transpose(pg, (0, 1, 3, 2, 4, 5)).reshape(Bb, _n_G * _k, _n_G * _k, 3)
    pgi = jnp.transpose(pg, (0, 3, 1, 2)).reshape(Bb, 3, _N_G * _K)
    pgi = jnp.transpose(pgi, (0, 2, 1))
    return pgi


# speedup vs baseline: 14.5182x; 1.1586x over previous
import functools

import jax
import jax.numpy as jnp
from jax import lax
from jax.experimental import pallas as pl
from jax.experimental.pallas import tpu as pltpu
from jax.experimental.pallas import tpu_sc as plsc

_N_G = 256
_N_C = 64
_K = 16
_k = 4
_n_G = 16
_B = 8
_N = 8192


def _patch_tail_pallas(pts_c, Wl1, bl1, Wl2, bl2):
    P = pts_c.shape[0] * _N_G
    PB = 256
    rows = PB * _N_C

    def body(x_ref, w1_ref, b1_ref, w2_ref, b2_ref, o_ref):
        x3 = x_ref[...].reshape(PB, _N_C, 3)
        center = jnp.mean(x3, axis=1, keepdims=True)
        cent = x3 - center
        nrm = jnp.sqrt(jnp.sum(cent * cent, axis=2, keepdims=True))
        scale = jnp.max(nrm, axis=1, keepdims=True) + 1e-9
        xn = (cent / scale).reshape(rows, 3)
        h = jnp.dot(xn, w1_ref[...], preferred_element_type=jnp.float32)
        h = jnp.maximum(h + b1_ref[...], 0.0)
        e = jnp.dot(h, w2_ref[...], preferred_element_type=jnp.float32) + b2_ref[...]
        e3 = e.reshape(PB, _N_C, 2)
        mn = jnp.min(e3, axis=1, keepdims=True)
        mx = jnp.max(e3, axis=1, keepdims=True)
        eps = 1e-6
        e3 = (e3 - mn) / (mx - mn + 1e-12) * (1.0 - 2.0 * eps) + eps

        exv = e3[:, :, 0].reshape(PB, 1, _N_C)
        eyv = e3[:, :, 1].reshape(PB, 1, _N_C)
        iota_k = lax.broadcasted_iota(jnp.int32, (PB, _K, 1), 1)
        gx = ((iota_k // _k).astype(jnp.float32) + 0.5) / _k
        gy = ((iota_k % _k).astype(jnp.float32) + 0.5) / _k
        gd = (gx - exv) ** 2 + (gy - eyv) ** 2
        gmn = jnp.min(gd, axis=2, keepdims=True)
        iota_c = lax.broadcasted_iota(jnp.int32, (PB, _K, _N_C), 2)
        sel = jnp.min(jnp.where(gd == gmn, iota_c, _N_C), axis=2, keepdims=True)
        oh = (iota_c == sel).astype(jnp.float32)
        out = []
        for c in range(3):
            pc = x3[:, :, c].reshape(PB, 1, _N_C)
            out.append(jnp.sum(oh * pc, axis=2))
        o_ref[...] = jnp.stack(out, axis=-1)

    return pl.pallas_call(
        body,
        grid=(P // PB,),
        in_specs=[
            pl.BlockSpec((rows, 3), lambda i: (i, 0)),
            pl.BlockSpec((3, 64), lambda i: (0, 0)),
            pl.BlockSpec((1, 64), lambda i: (0, 0)),
            pl.BlockSpec((64, 2), lambda i: (0, 0)),
            pl.BlockSpec((1, 2), lambda i: (0, 0)),
        ],
        out_specs=pl.BlockSpec((PB, _K, 3), lambda i: (i, 0, 0)),
        out_shape=jax.ShapeDtypeStruct((P, _K, 3), jnp.float32),
    )(pts_c.reshape(P * _N_C, 3), Wl1, bl1.reshape(1, -1), Wl2, bl2.reshape(1, -1))


def _mlp_pallas(x, W1, b1, W2, b2, row_block=None):
    R, Din = x.shape
    H = W1.shape[1]
    Dout = W2.shape[1]
    if row_block is None:
        row_block = R

    def body(x_ref, w1_ref, b1_ref, w2_ref, b2_ref, o_ref):
        h = jnp.dot(x_ref[...], w1_ref[...], preferred_element_type=jnp.float32)
        h = jnp.maximum(h + b1_ref[...], 0.0)
        o_ref[...] = (
            jnp.dot(h, w2_ref[...], preferred_element_type=jnp.float32) + b2_ref[...]
        )

    return pl.pallas_call(
        body,
        grid=(R // row_block,),
        in_specs=[
            pl.BlockSpec((row_block, Din), lambda i: (i, 0)),
            pl.BlockSpec((Din, H), lambda i: (0, 0)),
            pl.BlockSpec((1, H), lambda i: (0, 0)),
            pl.BlockSpec((H, Dout), lambda i: (0, 0)),
            pl.BlockSpec((1, Dout), lambda i: (0, 0)),
        ],
        out_specs=pl.BlockSpec((row_block, Dout), lambda i: (i, 0)),
        out_shape=jax.ShapeDtypeStruct((R, Dout), jnp.float32),
    )(x, W1, b1.reshape(1, -1), W2, b2.reshape(1, -1))


def _fps_recg_pallas(pts, Wg1, bg1, Wg2, bg2):
    Bb = pts.shape[0]
    R = _N // 128
    ptsr = jnp.transpose(pts, (0, 2, 1)).reshape(Bb, 3, R, 128)

    def body(p_ref, w1_ref, b1_ref, w2_ref, b2_ref, g_ref, r_ref):
        px = p_ref[:, 0]
        py = p_ref[:, 1]
        pz = p_ref[:, 2]
        flat = (
            jax.lax.broadcasted_iota(jnp.int32, (Bb, R, 128), 1) * 128
            + jax.lax.broadcasted_iota(jnp.int32, (Bb, R, 128), 2)
        )

        def step(i, carry):
            dists, lx, ly, lz = carry
            d = (px - lx) ** 2 + (py - ly) ** 2 + (pz - lz) ** 2
            dists = jnp.minimum(dists, d)
            m = jnp.max(dists, axis=(1, 2), keepdims=True)
            cand = jnp.where(dists == m, flat, _N)
            nxt = jnp.min(cand, axis=(1, 2), keepdims=True)
            sel = flat == nxt
            nlx = jnp.sum(jnp.where(sel, px, 0.0), axis=(1, 2), keepdims=True)
            nly = jnp.sum(jnp.where(sel, py, 0.0), axis=(1, 2), keepdims=True)
            nlz = jnp.sum(jnp.where(sel, pz, 0.0), axis=(1, 2), keepdims=True)
            row = jnp.concatenate([nlx, nly, nlz], axis=2)
            g_ref[:, pl.ds(i, 1), :] = row
            return dists, nlx, nly, nlz

        lx0 = px[:, 0:1, 0:1]
        ly0 = py[:, 0:1, 0:1]
        lz0 = pz[:, 0:1, 0:1]
        g_ref[:, 0:1, :] = jnp.concatenate([lx0, ly0, lz0], axis=2)
        dists0 = jnp.full((Bb, R, 128), 1e10, jnp.float32)
        jax.lax.fori_loop(1, _N_G, step, (dists0, lx0, ly0, lz0), unroll=2)

        ptsg = g_ref[...].reshape(Bb * _N_G, 3)
        h = jnp.dot(ptsg, w1_ref[...], preferred_element_type=jnp.float32)
        h = jnp.maximum(h + b1_ref[...], 0.0)
        rec = jnp.dot(h, w2_ref[...], preferred_element_type=jnp.float32) + b2_ref[...]
        r_ref[...] = rec.reshape(Bb, _N_G, 3)

    return pl.pallas_call(
        body,
        out_shape=(
            jax.ShapeDtypeStruct((Bb, _N_G, 3), jnp.float32),
            jax.ShapeDtypeStruct((Bb, _N_G, 3), jnp.float32),
        ),
    )(ptsr, Wg1, bg1.reshape(1, -1), Wg2, bg2.reshape(1, -1))


def _knn_gather_sc(ptsT, rec_gT):
    NW = 32
    Bh = ptsT.shape[0]
    WPB = NW // Bh
    APW = _N_G // WPB
    NCH = _N // 16
    CAP = _N + 64
    mesh = plsc.VectorSubcoreMesh(core_axis_name="c", subcore_axis_name="s")

    @functools.partial(
        pl.kernel,
        out_type=jax.ShapeDtypeStruct((Bh, _N_G, _N_C, 3), jnp.float32),
        mesh=mesh,
        compiler_params=pltpu.CompilerParams(
            use_tc_tiling_on_sc=False, needs_layout_passes=False
        ),
        scratch_types=[
            pltpu.VMEM((_N,), jnp.float32),
            pltpu.VMEM((_N,), jnp.float32),
            pltpu.VMEM((_N,), jnp.float32),
            pltpu.VMEM((_N,), jnp.float32),
            pltpu.VMEM((CAP,), jnp.int32),
            pltpu.VMEM((CAP,), jnp.int32),
            pltpu.VMEM((APW,), jnp.float32),
            pltpu.VMEM((APW,), jnp.float32),
            pltpu.VMEM((APW,), jnp.float32),
            pltpu.VMEM((_N_C + 16,), jnp.int32),
            pltpu.VMEM((APW, _N_C, 3), jnp.float32),
        ],
    )
    def knn(ptsT_hbm, recgT_hbm, out_hbm, px, py, pz, db, cd, ci, axr, ayr, azr, si, ob):
        cid = lax.axis_index("c")
        sid = lax.axis_index("s")
        w = cid * 16 + sid
        b = w // WPB
        a0 = (w % WPB) * APW
        pltpu.sync_copy(ptsT_hbm.at[b, 0], px)
        pltpu.sync_copy(ptsT_hbm.at[b, 1], py)
        pltpu.sync_copy(ptsT_hbm.at[b, 2], pz)
        pltpu.sync_copy(recgT_hbm.at[b, 0, pl.ds(a0, APW)], axr)
        pltpu.sync_copy(recgT_hbm.at[b, 1, pl.ds(a0, APW)], ayr)
        pltpu.sync_copy(recgT_hbm.at[b, 2, pl.ds(a0, APW)], azr)

        iota16 = lax.broadcasted_iota(jnp.int32, (16,), 0)
        inf16 = jnp.full((16,), jnp.inf, jnp.float32)
        K64 = _N_C

        def per_anchor(ai, _carry):
            base = (ai >> 4) << 4
            lane = ai & 15
            lsel = iota16 == lane
            axs = jnp.sum(jnp.where(lsel, axr[pl.ds(base, 16)], 0.0))
            ays = jnp.sum(jnp.where(lsel, ayr[pl.ds(base, 16)], 0.0))
            azs = jnp.sum(jnp.where(lsel, azr[pl.ds(base, 16)], 0.0))

            def p1(g, accs):
                o = g * 128
                new = list(accs)
                for r in range(8):
                    oo = o + r * 16
                    dx = px[pl.ds(oo, 16)] - axs
                    dy = py[pl.ds(oo, 16)] - ays
                    dz = pz[pl.ds(oo, 16)] - azs
                    dv = dx * dx + dy * dy + dz * dz
                    db[pl.ds(oo, 16)] = dv
                    new[r & 3] = jnp.minimum(new[r & 3], dv)
                return tuple(new)

            accs = lax.fori_loop(0, NCH // 8, p1, (inf16, inf16, inf16, inf16))
            maxv = jnp.maximum(jnp.maximum(accs[0], accs[1]),
                               jnp.maximum(accs[2], accs[3]))
            minv = jnp.minimum(jnp.minimum(accs[0], accs[1]),
                               jnp.minimum(accs[2], accs[3]))
            T = jnp.max(maxv)
            t_bits = jnp.max(plsc.bitcast(maxv, jnp.int32))
            lo_bits = jnp.min(plsc.bitcast(minv, jnp.int32))

            def p2(g, cnt):
                o = g * 64
                off = cnt
                for r in range(4):
                    oo = o + r * 16
                    dv = db[pl.ds(oo, 16)]
                    m = dv <= T
                    plsc.store_compressed(
                        cd.at[pl.ds(off, 16)], plsc.bitcast(dv, jnp.int32), mask=m
                    )
                    plsc.store_compressed(ci.at[pl.ds(off, 16)], iota16 + oo, mask=m)
                    off = off + jnp.sum(m.astype(jnp.int32))
                return off

            cnt = lax.fori_loop(0, NCH // 4, p2, jnp.int32(0))
            inf16i = jnp.full((16,), 0x7F800000, jnp.int32)
            cd[pl.ds(cnt, 16)] = inf16i
            cd[pl.ds(cnt + 16, 16)] = inf16i
            cd[pl.ds(cnt + 32, 16)] = inf16i
            cd[pl.ds(cnt + 48, 16)] = inf16i
            nc = (cnt + 15) >> 4
            nc4 = (cnt + 63) >> 6

            one16 = jnp.full((16,), 1, jnp.int32)
            zero16 = jnp.zeros((16,), jnp.int32)

            def count_le(mid):
                def cc(g, acc):
                    o = g * 64
                    for r in range(4):
                        acc = acc + jnp.where(
                            cd[pl.ds(o + r * 16, 16)] <= mid, one16, zero16
                        )
                    return acc
                return jnp.sum(lax.fori_loop(0, nc4, cc, zero16))

            def bs_cond(lohi):
                return lohi[0] < lohi[1]

            def bs(lohi):
                lo, hi = lohi
                mid = lo + ((hi - lo) >> 1)
                c = count_le(mid)
                return jnp.where(c >= K64, lo, mid + 1), jnp.where(c >= K64, mid, hi)

            lo, hi = lax.while_loop(bs_cond, bs, (lo_bits, t_bits))
            v = hi
            c_lt = count_le(v - 1)
            need = K64 - c_lt

            def p3(j, carry):
                nsel, eqbase = carry
                cv = cd[pl.ds(j * 16, 16)]
                m_lt = cv < v
                m_eq = cv == v
                eqrank = plsc.cumsum(m_eq.astype(jnp.int32)) + eqbase
                m_sel = m_lt | (m_eq & (eqrank <= need))
                plsc.store_compressed(
                    si.at[pl.ds(nsel, 16)], ci[pl.ds(j * 16, 16)], mask=m_sel
                )
                return (
                    nsel + jnp.sum(m_sel.astype(jnp.int32)),
                    eqbase + jnp.sum(m_eq.astype(jnp.int32)),
                )

            lax.fori_loop(0, nc, p3, (jnp.int32(0), jnp.int32(0)))

            ia16 = jnp.full((16,), 0, jnp.int32) + ai
            for t in range(K64 // 16):
                iv = si[pl.ds(t * 16, 16)]
                ipt = iota16 + t * 16
                plsc.store_scatter(
                    ob, [ia16, ipt, jnp.zeros((16,), jnp.int32)],
                    plsc.load_gather(px, [iv]),
                )
                plsc.store_scatter(
                    ob, [ia16, ipt, jnp.full((16,), 1, jnp.int32)],
                    plsc.load_gather(py, [iv]),
                )
                plsc.store_scatter(
                    ob, [ia16, ipt, jnp.full((16,), 2, jnp.int32)],
                    plsc.load_gather(pz, [iv]),
                )
            return 0

        lax.fori_loop(0, APW, per_anchor, 0)
        pltpu.sync_copy(ob, out_hbm.at[b, pl.ds(a0, APW)])

    return knn(ptsT, rec_gT)


def kernel(pts, Wg1, bg1, Wg2, bg2, Wl1, bl1, Wl2, bl2):
    Bb = pts.shape[0]
    Bh = Bb // 2
    halves = []
    for h in range(2):
        ph = pts[h * Bh:(h + 1) * Bh]
        pts_g, rec_g = _fps_recg_pallas(ph, Wg1, bg1, Wg2, bg2)
        ptsT = jnp.transpose(ph, (0, 2, 1))
        rec_gT = jnp.transpose(rec_g, (0, 2, 1))
        pts_c = _knn_gather_sc(ptsT, rec_gT)
        halves.append(_patch_tail_pallas(pts_c, Wl1, bl1, Wl2, bl2))

    pgi_local = jnp.concatenate(halves, axis=0)

    pg = pgi_local.reshape(Bb, _n_G, _n_G, _k, _k, 3)
    pg = jnp.transpose(pg, (0, 1, 3, 2, 4, 5)).reshape(Bb, _n_G * _k, _n_G * _k, 3)
    pgi = jnp.transpose(pg, (0, 3, 1, 2)).reshape(Bb, 3, _N_G * _K)
    pgi = jnp.transpose(pgi, (0, 2, 1))
    return pgi


# speedup vs baseline: 15.3856x; 1.0597x over previous
import functools

import jax
import jax.numpy as jnp
from jax import lax
from jax.experimental import pallas as pl
from jax.experimental.pallas import tpu as pltpu
from jax.experimental.pallas import tpu_sc as plsc

_N_G = 256
_N_C = 64
_K = 16
_k = 4
_n_G = 16
_B = 8
_N = 8192


def _patch_tail_pallas(pts_c, Wl1, bl1, Wl2, bl2):
    P = pts_c.shape[0] * _N_G
    PB = 256
    rows = PB * _N_C

    def body(x_ref, w1_ref, b1_ref, w2_ref, b2_ref, o_ref):
        x3 = x_ref[...].reshape(PB, _N_C, 3)
        center = jnp.mean(x3, axis=1, keepdims=True)
        cent = x3 - center
        nrm = jnp.sqrt(jnp.sum(cent * cent, axis=2, keepdims=True))
        scale = jnp.max(nrm, axis=1, keepdims=True) + 1e-9
        xn = (cent / scale).reshape(rows, 3)
        h = jnp.dot(xn, w1_ref[...], preferred_element_type=jnp.float32)
        h = jnp.maximum(h + b1_ref[...], 0.0)
        e = jnp.dot(h, w2_ref[...], preferred_element_type=jnp.float32) + b2_ref[...]
        e3 = e.reshape(PB, _N_C, 2)
        mn = jnp.min(e3, axis=1, keepdims=True)
        mx = jnp.max(e3, axis=1, keepdims=True)
        eps = 1e-6
        e3 = (e3 - mn) / (mx - mn + 1e-12) * (1.0 - 2.0 * eps) + eps

        exv = e3[:, :, 0].reshape(PB, 1, _N_C)
        eyv = e3[:, :, 1].reshape(PB, 1, _N_C)
        iota_k = lax.broadcasted_iota(jnp.int32, (PB, _K, 1), 1)
        gx = ((iota_k // _k).astype(jnp.float32) + 0.5) / _k
        gy = ((iota_k % _k).astype(jnp.float32) + 0.5) / _k
        gd = (gx - exv) ** 2 + (gy - eyv) ** 2
        gmn = jnp.min(gd, axis=2, keepdims=True)
        iota_c = lax.broadcasted_iota(jnp.int32, (PB, _K, _N_C), 2)
        sel = jnp.min(jnp.where(gd == gmn, iota_c, _N_C), axis=2, keepdims=True)
        oh = (iota_c == sel).astype(jnp.float32)
        out = []
        for c in range(3):
            pc = x3[:, :, c].reshape(PB, 1, _N_C)
            out.append(jnp.sum(oh * pc, axis=2))
        o_ref[...] = jnp.stack(out, axis=-1)

    return pl.pallas_call(
        body,
        grid=(P // PB,),
        in_specs=[
            pl.BlockSpec((rows, 3), lambda i: (i, 0)),
            pl.BlockSpec((3, 64), lambda i: (0, 0)),
            pl.BlockSpec((1, 64), lambda i: (0, 0)),
            pl.BlockSpec((64, 2), lambda i: (0, 0)),
            pl.BlockSpec((1, 2), lambda i: (0, 0)),
        ],
        out_specs=pl.BlockSpec((PB, _K, 3), lambda i: (i, 0, 0)),
        out_shape=jax.ShapeDtypeStruct((P, _K, 3), jnp.float32),
    )(pts_c.reshape(P * _N_C, 3), Wl1, bl1.reshape(1, -1), Wl2, bl2.reshape(1, -1))


def _mlp_pallas(x, W1, b1, W2, b2, row_block=None):
    R, Din = x.shape
    H = W1.shape[1]
    Dout = W2.shape[1]
    if row_block is None:
        row_block = R

    def body(x_ref, w1_ref, b1_ref, w2_ref, b2_ref, o_ref):
        h = jnp.dot(x_ref[...], w1_ref[...], preferred_element_type=jnp.float32)
        h = jnp.maximum(h + b1_ref[...], 0.0)
        o_ref[...] = (
            jnp.dot(h, w2_ref[...], preferred_element_type=jnp.float32) + b2_ref[...]
        )

    return pl.pallas_call(
        body,
        grid=(R // row_block,),
        in_specs=[
            pl.BlockSpec((row_block, Din), lambda i: (i, 0)),
            pl.BlockSpec((Din, H), lambda i: (0, 0)),
            pl.BlockSpec((1, H), lambda i: (0, 0)),
            pl.BlockSpec((H, Dout), lambda i: (0, 0)),
            pl.BlockSpec((1, Dout), lambda i: (0, 0)),
        ],
        out_specs=pl.BlockSpec((row_block, Dout), lambda i: (i, 0)),
        out_shape=jax.ShapeDtypeStruct((R, Dout), jnp.float32),
    )(x, W1, b1.reshape(1, -1), W2, b2.reshape(1, -1))


def _fps_recg_pallas(pts, Wg1, bg1, Wg2, bg2):
    Bb = pts.shape[0]
    R = _N // 128
    ptsr = jnp.transpose(pts, (0, 2, 1)).reshape(Bb, 3, R, 128)

    def body(p_ref, w1_ref, b1_ref, w2_ref, b2_ref, g_ref, r_ref):
        px = p_ref[:, 0]
        py = p_ref[:, 1]
        pz = p_ref[:, 2]
        flat = (
            jax.lax.broadcasted_iota(jnp.int32, (Bb, R, 128), 1) * 128
            + jax.lax.broadcasted_iota(jnp.int32, (Bb, R, 128), 2)
        )

        def step(i, carry):
            dists, lx, ly, lz = carry
            d = (px - lx) ** 2 + (py - ly) ** 2 + (pz - lz) ** 2
            dists = jnp.minimum(dists, d)
            m = jnp.max(dists, axis=(1, 2), keepdims=True)
            cand = jnp.where(dists == m, flat, _N)
            nxt = jnp.min(cand, axis=(1, 2), keepdims=True)
            sel = flat == nxt
            nlx = jnp.sum(jnp.where(sel, px, 0.0), axis=(1, 2), keepdims=True)
            nly = jnp.sum(jnp.where(sel, py, 0.0), axis=(1, 2), keepdims=True)
            nlz = jnp.sum(jnp.where(sel, pz, 0.0), axis=(1, 2), keepdims=True)
            row = jnp.concatenate([nlx, nly, nlz], axis=2)
            g_ref[:, pl.ds(i, 1), :] = row
            return dists, nlx, nly, nlz

        lx0 = px[:, 0:1, 0:1]
        ly0 = py[:, 0:1, 0:1]
        lz0 = pz[:, 0:1, 0:1]
        g_ref[:, 0:1, :] = jnp.concatenate([lx0, ly0, lz0], axis=2)
        dists0 = jnp.full((Bb, R, 128), 1e10, jnp.float32)
        jax.lax.fori_loop(1, _N_G, step, (dists0, lx0, ly0, lz0), unroll=2)

        ptsg = g_ref[...].reshape(Bb * _N_G, 3)
        h = jnp.dot(ptsg, w1_ref[...], preferred_element_type=jnp.float32)
        h = jnp.maximum(h + b1_ref[...], 0.0)
        rec = jnp.dot(h, w2_ref[...], preferred_element_type=jnp.float32) + b2_ref[...]
        r_ref[...] = rec.reshape(Bb, _N_G, 3)

    return pl.pallas_call(
        body,
        out_shape=(
            jax.ShapeDtypeStruct((Bb, _N_G, 3), jnp.float32),
            jax.ShapeDtypeStruct((Bb, _N_G, 3), jnp.float32),
        ),
    )(ptsr, Wg1, bg1.reshape(1, -1), Wg2, bg2.reshape(1, -1))


def _knn_gather_sc(ptsT, rec_gT):
    NW = 32
    Bh = ptsT.shape[0]
    WPB = NW // Bh
    APW = _N_G // WPB
    NCH = _N // 16
    CAP = _N + 64
    mesh = plsc.VectorSubcoreMesh(core_axis_name="c", subcore_axis_name="s")

    @functools.partial(
        pl.kernel,
        out_type=jax.ShapeDtypeStruct((Bh, _N_G, _N_C, 3), jnp.float32),
        mesh=mesh,
        compiler_params=pltpu.CompilerParams(
            use_tc_tiling_on_sc=False, needs_layout_passes=False
        ),
        scratch_types=[
            pltpu.VMEM((_N,), jnp.float32),
            pltpu.VMEM((_N,), jnp.float32),
            pltpu.VMEM((_N,), jnp.float32),
            pltpu.VMEM((_N,), jnp.float32),
            pltpu.VMEM((CAP,), jnp.int32),
            pltpu.VMEM((CAP,), jnp.int32),
            pltpu.VMEM((APW,), jnp.float32),
            pltpu.VMEM((APW,), jnp.float32),
            pltpu.VMEM((APW,), jnp.float32),
            pltpu.VMEM((_N_C + 16,), jnp.int32),
            pltpu.VMEM((APW, _N_C, 3), jnp.float32),
        ],
    )
    def knn(ptsT_hbm, recgT_hbm, out_hbm, px, py, pz, db, cd, ci, axr, ayr, azr, si, ob):
        cid = lax.axis_index("c")
        sid = lax.axis_index("s")
        w = cid * 16 + sid
        b = w // WPB
        a0 = (w % WPB) * APW
        pltpu.sync_copy(ptsT_hbm.at[b, 0], px)
        pltpu.sync_copy(ptsT_hbm.at[b, 1], py)
        pltpu.sync_copy(ptsT_hbm.at[b, 2], pz)
        pltpu.sync_copy(recgT_hbm.at[b, 0, pl.ds(a0, APW)], axr)
        pltpu.sync_copy(recgT_hbm.at[b, 1, pl.ds(a0, APW)], ayr)
        pltpu.sync_copy(recgT_hbm.at[b, 2, pl.ds(a0, APW)], azr)

        iota16 = lax.broadcasted_iota(jnp.int32, (16,), 0)
        inf16 = jnp.full((16,), jnp.inf, jnp.float32)
        K64 = _N_C

        def per_anchor(ai, _carry):
            base = (ai >> 4) << 4
            lane = ai & 15
            lsel = iota16 == lane
            axs = jnp.sum(jnp.where(lsel, axr[pl.ds(base, 16)], 0.0))
            ays = jnp.sum(jnp.where(lsel, ayr[pl.ds(base, 16)], 0.0))
            azs = jnp.sum(jnp.where(lsel, azr[pl.ds(base, 16)], 0.0))

            def p1(g, accs):
                o = g * 128
                new = list(accs)
                for r in range(8):
                    oo = o + r * 16
                    dx = px[pl.ds(oo, 16)] - axs
                    dy = py[pl.ds(oo, 16)] - ays
                    dz = pz[pl.ds(oo, 16)] - azs
                    dv = dx * dx + dy * dy + dz * dz
                    db[pl.ds(oo, 16)] = dv
                    new[r & 3] = jnp.minimum(new[r & 3], dv)
                return tuple(new)

            accs = lax.fori_loop(0, NCH // 8, p1, (inf16, inf16, inf16, inf16))
            maxv = jnp.maximum(jnp.maximum(accs[0], accs[1]),
                               jnp.maximum(accs[2], accs[3]))
            minv = jnp.minimum(jnp.minimum(accs[0], accs[1]),
                               jnp.minimum(accs[2], accs[3]))
            T = jnp.max(maxv)
            t_bits = jnp.max(plsc.bitcast(maxv, jnp.int32))
            lo_bits = jnp.min(plsc.bitcast(minv, jnp.int32))

            def p2(g, cnt):
                o = g * 64
                off = cnt
                for r in range(4):
                    oo = o + r * 16
                    dv = db[pl.ds(oo, 16)]
                    m = dv <= T
                    plsc.store_compressed(
                        cd.at[pl.ds(off, 16)], plsc.bitcast(dv, jnp.int32), mask=m
                    )
                    plsc.store_compressed(ci.at[pl.ds(off, 16)], iota16 + oo, mask=m)
                    off = off + jnp.sum(m.astype(jnp.int32))
                return off

            cnt = lax.fori_loop(0, NCH // 4, p2, jnp.int32(0))
            inf16i = jnp.full((16,), 0x7F800000, jnp.int32)
            cd[pl.ds(cnt, 16)] = inf16i
            cd[pl.ds(cnt + 16, 16)] = inf16i
            cd[pl.ds(cnt + 32, 16)] = inf16i
            cd[pl.ds(cnt + 48, 16)] = inf16i
            nc = (cnt + 15) >> 4
            nc4 = (cnt + 63) >> 6

            one16 = jnp.full((16,), 1, jnp.int32)
            zero16 = jnp.zeros((16,), jnp.int32)

            def count_le(mid):
                def cc(g, acc):
                    o = g * 64
                    for r in range(4):
                        acc = acc + jnp.where(
                            cd[pl.ds(o + r * 16, 16)] <= mid, one16, zero16
                        )
                    return acc
                return jnp.sum(lax.fori_loop(0, nc4, cc, zero16))

            def bs_cond(lohi):
                return lohi[0] < lohi[1]

            def bs(lohi):
                lo, hi = lohi
                mid = lo + ((hi - lo) >> 1)
                c = count_le(mid)
                return jnp.where(c >= K64, lo, mid + 1), jnp.where(c >= K64, mid, hi)

            lo, hi = lax.while_loop(bs_cond, bs, (lo_bits, t_bits))
            v = hi
            c_lt = count_le(v - 1)
            need = K64 - c_lt

            def p3(j, carry):
                nsel, eqbase = carry
                cv = cd[pl.ds(j * 16, 16)]
                m_lt = cv < v
                m_eq = cv == v
                eqrank = plsc.cumsum(m_eq.astype(jnp.int32)) + eqbase
                m_sel = m_lt | (m_eq & (eqrank <= need))
                plsc.store_compressed(
                    si.at[pl.ds(nsel, 16)], ci[pl.ds(j * 16, 16)], mask=m_sel
                )
                return (
                    nsel + jnp.sum(m_sel.astype(jnp.int32)),
                    eqbase + jnp.sum(m_eq.astype(jnp.int32)),
                )

            lax.fori_loop(0, nc, p3, (jnp.int32(0), jnp.int32(0)))

            ia16 = jnp.full((16,), 0, jnp.int32) + ai
            for t in range(K64 // 16):
                iv = si[pl.ds(t * 16, 16)]
                ipt = iota16 + t * 16
                plsc.store_scatter(
                    ob, [ia16, ipt, jnp.zeros((16,), jnp.int32)],
                    plsc.load_gather(px, [iv]),
                )
                plsc.store_scatter(
                    ob, [ia16, ipt, jnp.full((16,), 1, jnp.int32)],
                    plsc.load_gather(py, [iv]),
                )
                plsc.store_scatter(
                    ob, [ia16, ipt, jnp.full((16,), 2, jnp.int32)],
                    plsc.load_gather(pz, [iv]),
                )
            return 0

        lax.fori_loop(0, APW, per_anchor, 0)
        pltpu.sync_copy(ob, out_hbm.at[b, pl.ds(a0, APW)])

    return knn(ptsT, rec_gT)


def kernel(pts, Wg1, bg1, Wg2, bg2, Wl1, bl1, Wl2, bl2):
    Bb = pts.shape[0]
    NS = 4
    Bh = Bb // NS
    halves = []
    for h in range(NS):
        ph = pts[h * Bh:(h + 1) * Bh]
        pts_g, rec_g = _fps_recg_pallas(ph, Wg1, bg1, Wg2, bg2)
        ptsT = jnp.transpose(ph, (0, 2, 1))
        rec_gT = jnp.transpose(rec_g, (0, 2, 1))
        pts_c = _knn_gather_sc(ptsT, rec_gT)
        halves.append(_patch_tail_pallas(pts_c, Wl1, bl1, Wl2, bl2))

    pgi_local = jnp.concatenate(halves, axis=0)

    pg = pgi_local.reshape(Bb, _n_G, _n_G, _k, _k, 3)
    pg = jnp.transpose(pg, (0, 1, 3, 2, 4, 5)).reshape(Bb, _n_G * _k, _n_G * _k, 3)
    pgi = jnp.transpose(pg, (0, 3, 1, 2)).reshape(Bb, 3, _N_G * _K)
    pgi = jnp.transpose(pgi, (0, 2, 1))
    return pgi
